# Initial kernel scaffold; baseline (speedup 1.0000x reference)
import jax
import jax.numpy as jnp
from jax.experimental import pallas as pl


def kernel(x, edge_attr, nA, nB, system_size, params, edge_index, batch):
    raise NotImplementedError("write your pallas kernel here")


# speedup vs baseline: 1.0035x; 1.0035x over previous
import functools

import jax
import jax.numpy as jnp
import numpy as np
from jax.experimental import pallas as pl

N_NODES = 50000
N_PAD = 50048
N_EDGES = 800000
HID = 64
NUM_LAYERS = 10
NUM_GRAPHS = 16


def _ln(x, g, b, eps=1e-5):
    m = jnp.mean(x, axis=-1, keepdims=True)
    v = jnp.mean((x - m) ** 2, axis=-1, keepdims=True)
    return (x - m) / jnp.sqrt(v + eps) * g + b


def _silu(x):
    return x * jax.nn.sigmoid(x)


def _softplus(x):
    return jnp.maximum(x, 0.0) + jnp.log1p(jnp.exp(-jnp.abs(x)))


_S2S_CHUNK = 2944
_S2S_NCHUNK = N_PAD // _S2S_CHUNK


def _s2s_head_body(*refs):
    (h_ref, oh_ref, gf_ref,
     wih0, whh0, bih0, bhh0,
     wih1, whh1, bih1, bhh1,
     rp_w, rp_b, rp_g, rp_b2,
     gm_w1, gm_b1, gm_g, gm_bln, gm_w2, gm_b2,
     fm_w1, fm_b1, fm_g1, fm_bln1,
     fm_w2, fm_b2, fm_g2, fm_bln2,
     fm_w3, fm_b3,
     out_ref) = refs

    B = NUM_GRAPHS
    H = HID
    CH = _S2S_CHUNK

    def one_s2s(wih, whh, bih, bhh):
        hst = jnp.zeros((B, H), jnp.float32)
        cst = jnp.zeros((B, H), jnp.float32)
        qs = jnp.zeros((B, 2 * H), jnp.float32)
        for _ in range(4):
            z = (jnp.dot(qs, wih[...], preferred_element_type=jnp.float32)
                 + bih[...]
                 + jnp.dot(hst, whh[...], preferred_element_type=jnp.float32)
                 + bhh[...])
            i_g, f_g, g_g, o_g = jnp.split(z, 4, axis=-1)
            cst = jax.nn.sigmoid(f_g) * cst + jax.nn.sigmoid(i_g) * jnp.tanh(g_g)
            hst = jax.nn.sigmoid(o_g) * jnp.tanh(cst)
            def max_step(c, emax):
                xc = h_ref[pl.ds(c * CH, CH), :]
                mc = oh_ref[:, pl.ds(c * CH, CH)]
                e0 = jax.lax.dot_general(
                    hst, xc, (((1,), (1,)), ((), ())),
                    preferred_element_type=jnp.float32)
                emsk = jnp.where(mc > 0.0, e0, -jnp.inf)
                return jnp.maximum(emax, jnp.max(emsk, axis=1, keepdims=True))
            emax = jax.lax.fori_loop(
                0, _S2S_NCHUNK, max_step,
                jnp.full((B, 1), -jnp.inf, jnp.float32))
            emax = jnp.where(jnp.isfinite(emax), emax, 0.0)

            def acc_step(c, carry):
                den, num = carry
                xc = h_ref[pl.ds(c * CH, CH), :]
                mc = oh_ref[:, pl.ds(c * CH, CH)]
                e0 = jax.lax.dot_general(
                    hst, xc, (((1,), (1,)), ((), ())),
                    preferred_element_type=jnp.float32)
                eng = jnp.sum(e0 * mc, axis=0, keepdims=True)
                esel = jnp.sum(mc * emax, axis=0, keepdims=True)
                ex = jnp.exp(eng - esel)
                w = mc * ex
                den = den + jnp.sum(w, axis=1, keepdims=True)
                num = num + jnp.dot(w, xc,
                                    preferred_element_type=jnp.float32)
                return den, num
            den, num = jax.lax.fori_loop(
                0, _S2S_NCHUNK, acc_step,
                (jnp.zeros((B, 1), jnp.float32), jnp.zeros((B, H), jnp.float32)))
            r = num / (den + 1e-16)
            qs = jnp.concatenate([hst, r], axis=1)
        return qs

    qs0 = one_s2s(wih0, whh0, bih0, bhh0)
    qs1 = one_s2s(wih1, whh1, bih1, bhh1)
    hr = jnp.concatenate([qs0, qs1], axis=1)
    hr = _silu(_ln(jnp.dot(hr, rp_w[...], preferred_element_type=jnp.float32)
                   + rp_b[...], rp_g[...], rp_b2[...]))
    g = _silu(_ln(jnp.dot(gf_ref[...], gm_w1[...], preferred_element_type=jnp.float32)
                  + gm_b1[...], gm_g[...], gm_bln[...]))
    g = jnp.dot(g, gm_w2[...], preferred_element_type=jnp.float32) + gm_b2[...]
    comb = jnp.concatenate([hr, g], axis=1)
    f = _silu(_ln(jnp.dot(comb, fm_w1[...], preferred_element_type=jnp.float32)
                  + fm_b1[...], fm_g1[...], fm_bln1[...]))
    f = _silu(_ln(jnp.dot(f, fm_w2[...], preferred_element_type=jnp.float32)
                  + fm_b2[...], fm_g2[...], fm_bln2[...]))
    f = jnp.dot(f, fm_w3[...], preferred_element_type=jnp.float32) + fm_b3[...]
    out_ref[...] = _softplus(f).T


def _s2s_head(h_pad, oh, gf, params):
    s2s = params['s2s']
    rp = params['rp']
    gm = params['gm']
    fm = params['fm']
    r2 = lambda v: v.reshape(1, -1)
    args = [
        h_pad, oh, gf,
        s2s[0]['Wih'], s2s[0]['Whh'], r2(s2s[0]['bih']), r2(s2s[0]['bhh']),
        s2s[1]['Wih'], s2s[1]['Whh'], r2(s2s[1]['bih']), r2(s2s[1]['bhh']),
        rp['lin']['W'], r2(rp['lin']['b']), r2(rp['ln']['g']), r2(rp['ln']['b']),
        gm['lin1']['W'], r2(gm['lin1']['b']), r2(gm['ln']['g']), r2(gm['ln']['b']),
        gm['lin2']['W'], r2(gm['lin2']['b']),
        fm['lin1']['W'], r2(fm['lin1']['b']), r2(fm['ln1']['g']), r2(fm['ln1']['b']),
        fm['lin2']['W'], r2(fm['lin2']['b']), r2(fm['ln2']['g']), r2(fm['ln2']['b']),
        fm['lin3']['W'], r2(fm['lin3']['b']),
    ]
    out = pl.pallas_call(
        _s2s_head_body,
        out_shape=jax.ShapeDtypeStruct((1, NUM_GRAPHS), jnp.float32),
    )(*args)
    return out.reshape(NUM_GRAPHS)


def _ap(x, l):
    return x @ l['W'] + l['b']


def _aln(x, l):
    return _ln(x, l['g'], l['b'])


def _gine(h, src, dst, e, p):
    ee = _ap(e, p['lin_edge'])
    msg = jax.nn.relu(h[src] + ee)
    agg = jax.ops.segment_sum(msg, dst, num_segments=h.shape[0])
    z = h + agg
    z = _silu(_aln(_ap(z, p['mlp1']), p['mlp_ln']))
    z = _ap(z, p['mlp2'])
    return z


def _tconv(h, src, dst, e, p):
    N = h.shape[0]
    H = 8
    C = HID // 8
    q = _ap(h, p['q']).reshape(N, H, C)
    k = _ap(h, p['k']).reshape(N, H, C)
    v = _ap(h, p['v']).reshape(N, H, C)
    ee = _ap(e, p['e']).reshape(-1, H, C)
    kj = k[src] + ee
    alpha = jnp.sum(q[dst] * kj, axis=-1) / np.sqrt(C)
    amax = jax.ops.segment_max(alpha, dst, num_segments=N)
    amax = jnp.where(jnp.isfinite(amax), amax, 0.0)
    ex = jnp.exp(alpha - amax[dst])
    den = jax.ops.segment_sum(ex, dst, num_segments=N)
    a = ex / (den[dst] + 1e-16)
    out = jax.ops.segment_sum((v[src] + ee) * a[:, :, None], dst,
                              num_segments=N).reshape(N, HID)
    xr = _ap(h, p['skip'])
    beta = jax.nn.sigmoid(jnp.concatenate([out, xr, out - xr], axis=-1) @ p['Wbeta'])
    return beta * xr + (1.0 - beta) * out


def kernel(x, edge_attr, nA, nB, system_size, params, edge_index, batch):
    src, dst = edge_index[0], edge_index[1]
    node_features = x
    edge_features = edge_attr
    h = _silu(_aln(_ap(node_features, params['node_enc']['lin']),
                   params['node_enc']['ln']))
    e = _silu(_aln(_ap(edge_features, params['edge_enc']['lin']),
                   params['edge_enc']['ln']))
    for i in range(NUM_LAYERS):
        lp = params['layers'][i]
        e = _silu(_aln(_ap(e, lp['ec_lin']), lp['ec_ln']))
        if i % 2 == 0:
            hn = _gine(h, src, dst, e, lp['gine'])
        else:
            hn = _tconv(h, src, dst, e, lp['tc'])
        hn = _aln(hn, lp['norm'])
        h = h + hn
        if i % 2 == 0 and i // 2 < NUM_LAYERS // 2:
            pp = params['pool'][i // 2]
            h = _silu(_aln(_ap(h, pp['lin']), pp['ln']))

    h_pad = jnp.pad(h, ((0, N_PAD - N_NODES), (0, 0)))
    gids = jnp.arange(NUM_GRAPHS, dtype=jnp.int32)
    batch_pad = jnp.pad(batch, (0, N_PAD - N_NODES), constant_values=-1)
    oh = (batch_pad[None, :] == gids[:, None]).astype(jnp.float32)
    nAn = nA[:, 0] / (system_size[:, 0] + 1e-10)
    nBn = nB[:, 0] / (system_size[:, 0] + 1e-10)
    gf = jnp.stack([nAn, nBn], axis=1)
    return _s2s_head(h_pad, oh, gf, params)


# speedup vs baseline: 14.3250x; 14.2749x over previous
import functools

import jax
import jax.numpy as jnp
import numpy as np
from jax import lax
from jax.experimental import pallas as pl
from jax.experimental.pallas import tpu as pltpu
from jax.experimental.pallas import tpu_sc as plsc

N_NODES = 50000
N_PAD = 50048
N_EDGES = 800000
EP = 802816
HID = 64
NUM_LAYERS = 10
NUM_GRAPHS = 16

_EB = 2048
_NB = 3128

_mesh_cache = []


def _mesh():
    if not _mesh_cache:
        _mesh_cache.append(plsc.VectorSubcoreMesh(
            core_axis_name="c", subcore_axis_name="s"))
    return _mesh_cache[0]


def _ln(x, g, b, eps=1e-5):
    m = jnp.mean(x, axis=-1, keepdims=True)
    v = jnp.mean((x - m) ** 2, axis=-1, keepdims=True)
    return (x - m) / jnp.sqrt(v + eps) * g + b


def _silu(x):
    return x * jax.nn.sigmoid(x)


def _softplus(x):
    return jnp.maximum(x, 0.0) + jnp.log1p(jnp.exp(-jnp.abs(x)))


def _dot(a, b):
    return jnp.dot(a, b, preferred_element_type=jnp.float32)


_OPS_W = EP // 128 // 32
_OPS_T = EP // 128 // 16
_TROWS = N_PAD // 16


def _sc_gather(table, idx2d):
    @functools.partial(
        pl.kernel, mesh=_mesh(),
        compiler_params=pltpu.CompilerParams(use_tc_tiling_on_sc=False),
        out_type=jax.ShapeDtypeStruct((EP, 64), jnp.float32),
        scratch_types=[
            pltpu.VMEM((_OPS_W, 128), jnp.int32),
            pltpu.VMEM((128, 64), jnp.float32),
            pltpu.VMEM((128, 64), jnp.float32),
            pltpu.SemaphoreType.DMA,
            pltpu.SemaphoreType.DMA,
        ],
    )
    def k(table_hbm, idx_hbm, out_hbm, idx_v, buf0, buf1, s0, s1):
        cid = lax.axis_index("c")
        sid = lax.axis_index("s")
        wid = sid * 2 + cid
        pltpu.sync_copy(idx_hbm.at[wid], idx_v)
        ebase = wid * _OPS_W * 128
        pltpu.async_copy(table_hbm.at[idx_v.at[0]], buf0, s0)
        pltpu.async_copy(table_hbm.at[idx_v.at[1]], buf1, s1)

        def body(jj, carry):
            op0 = jj * 2
            pltpu.make_async_copy(table_hbm.at[idx_v.at[0]], buf0, s0).wait()
            pltpu.sync_copy(buf0, out_hbm.at[pl.ds(ebase + op0 * 128, 128), :])

            @pl.when(op0 + 2 < _OPS_W)
            def _():
                pltpu.async_copy(table_hbm.at[idx_v.at[op0 + 2]], buf0, s0)

            op1 = op0 + 1
            pltpu.make_async_copy(table_hbm.at[idx_v.at[0]], buf1, s1).wait()
            pltpu.sync_copy(buf1, out_hbm.at[pl.ds(ebase + op1 * 128, 128), :])

            @pl.when(op1 + 2 < _OPS_W)
            def _():
                pltpu.async_copy(table_hbm.at[idx_v.at[op1 + 2]], buf1, s1)

            return carry

        lax.fori_loop(0, _OPS_W // 2, body, 0)

    return k(table, idx2d)


def _sc_scatter_add(msg01, dst3s, zeros32):
    @functools.partial(
        pl.kernel, mesh=_mesh(),
        compiler_params=pltpu.CompilerParams(use_tc_tiling_on_sc=False),
        out_type=jax.ShapeDtypeStruct((2, N_PAD, 32), jnp.float32),
        scratch_types=[
            pltpu.VMEM((8, 128), jnp.int32),
            pltpu.VMEM((128, 32), jnp.float32),
            pltpu.VMEM_SHARED((N_PAD, 32), jnp.float32),
        ],
    )
    def k(msg_hbm, idx_hbm, z_hbm, out_hbm, idx_v, buf, acc):
        cid = lax.axis_index("c")
        sid = lax.axis_index("s")

        @pl.when(sid == 0)
        def _():
            pltpu.sync_copy(z_hbm, acc)

        plsc.subcore_barrier()

        def outer(j8, carry):
            pltpu.sync_copy(idx_hbm.at[sid, pl.ds(j8 * 8, 8), :], idx_v)

            def body(j, carry2):
                op = j8 * 8 + j
                pltpu.sync_copy(
                    msg_hbm.at[cid,
                               pl.ds(sid * _OPS_T * 128 + op * 128, 128), :],
                    buf)
                pltpu.sync_copy(buf, acc.at[idx_v.at[j]], add=True)
                return carry2

            return lax.fori_loop(0, 8, body, carry)

        lax.fori_loop(0, _OPS_T // 8, outer, 0)
        plsc.subcore_barrier()
        pltpu.sync_copy(acc.at[pl.ds(sid * _TROWS, _TROWS), :],
                        out_hbm.at[cid, pl.ds(sid * _TROWS, _TROWS), :])

    return k(msg01, dst3s, zeros32)


def _eblk(i):
    return (i, 0)


def _edge_specs(n):
    return [pl.BlockSpec((_EB, 64), _eblk) for _ in range(n)]


def _w_spec(shape):
    return pl.BlockSpec(shape, lambda i: tuple(0 for _ in shape))


def _split_spec():
    return pl.BlockSpec((2, _EB, 32), lambda i: (0, i, 0))


def _nsplit_spec():
    return pl.BlockSpec((2, _NB, 32), lambda i: (0, i, 0))


def _enc_body(x_ref, w, b, g, bb, out_ref):
    z = _ln(_dot(x_ref[...], w[...]) + b[...], g[...], bb[...])
    out_ref[...] = _silu(z)


def _encode_nodes(x_pad, p):
    return pl.pallas_call(
        _enc_body,
        grid=(N_PAD // _NB,),
        in_specs=[pl.BlockSpec((_NB, 4), _eblk), _w_spec((4, HID)),
                  _w_spec((1, HID)), _w_spec((1, HID)), _w_spec((1, HID))],
        out_specs=pl.BlockSpec((_NB, 64), _eblk),
        out_shape=jax.ShapeDtypeStruct((N_PAD, 64), jnp.float32),
    )(x_pad, p['lin']['W'], p['lin']['b'].reshape(1, -1),
      p['ln']['g'].reshape(1, -1), p['ln']['b'].reshape(1, -1))


def _encode_edges(ea_pad, p):
    return pl.pallas_call(
        _enc_body,
        grid=(EP // _EB,),
        in_specs=[pl.BlockSpec((_EB, 3), _eblk), _w_spec((3, HID)),
                  _w_spec((1, HID)), _w_spec((1, HID)), _w_spec((1, HID))],
        out_specs=pl.BlockSpec((_EB, 64), _eblk),
        out_shape=jax.ShapeDtypeStruct((EP, 64), jnp.float32),
    )(ea_pad, p['lin']['W'], p['lin']['b'].reshape(1, -1),
      p['ln']['g'].reshape(1, -1), p['ln']['b'].reshape(1, -1))


def _gine_edge_body(e_ref, hs_ref, w1, b1, g1, bb1, w2, b2,
                    eo_ref, msg_ref):
    ep = _silu(_ln(_dot(e_ref[...], w1[...]) + b1[...], g1[...], bb1[...]))
    eo_ref[...] = ep
    ee = _dot(ep, w2[...]) + b2[...]
    m = jnp.maximum(hs_ref[...] + ee, 0.0)
    msg_ref[0] = m[:, :32]
    msg_ref[1] = m[:, 32:]


def _gine_edge(e, hs, lp):
    gp = lp['gine']
    return pl.pallas_call(
        _gine_edge_body,
        grid=(EP // _EB,),
        in_specs=_edge_specs(2) + [
            _w_spec((64, 64)), _w_spec((1, 64)), _w_spec((1, 64)),
            _w_spec((1, 64)), _w_spec((64, 64)), _w_spec((1, 64))],
        out_specs=[pl.BlockSpec((_EB, 64), _eblk), _split_spec()],
        out_shape=[jax.ShapeDtypeStruct((EP, 64), jnp.float32),
                   jax.ShapeDtypeStruct((2, EP, 32), jnp.float32)],
    )(e, hs,
      lp['ec_lin']['W'], lp['ec_lin']['b'].reshape(1, -1),
      lp['ec_ln']['g'].reshape(1, -1), lp['ec_ln']['b'].reshape(1, -1),
      gp['lin_edge']['W'], gp['lin_edge']['b'].reshape(1, -1))


def _gine_node_body(h_ref, agg_ref, m1w, m1b, mlg, mlb, m2w, m2b,
                    ng, nb, pw, pb, pg, pbb, out_ref):
    h = h_ref[...]
    agg = jnp.concatenate([agg_ref[0], agg_ref[1]], axis=1)
    z = h + agg
    z = _silu(_ln(_dot(z, m1w[...]) + m1b[...], mlg[...], mlb[...]))
    z = _dot(z, m2w[...]) + m2b[...]
    hn = _ln(z, ng[...], nb[...])
    h2 = h + hn
    h2 = _silu(_ln(_dot(h2, pw[...]) + pb[...], pg[...], pbb[...]))
    out_ref[...] = h2


def _gine_node(h, agg01, lp, pp):
    gp = lp['gine']
    r2 = lambda v: v.reshape(1, -1)
    return pl.pallas_call(
        _gine_node_body,
        grid=(N_PAD // _NB,),
        in_specs=[pl.BlockSpec((_NB, 64), _eblk), _nsplit_spec(),
                  _w_spec((64, 64)), _w_spec((1, 64)), _w_spec((1, 64)),
                  _w_spec((1, 64)), _w_spec((64, 64)), _w_spec((1, 64)),
                  _w_spec((1, 64)), _w_spec((1, 64)),
                  _w_spec((64, 64)), _w_spec((1, 64)), _w_spec((1, 64)),
                  _w_spec((1, 64))],
        out_specs=pl.BlockSpec((_NB, 64), _eblk),
        out_shape=jax.ShapeDtypeStruct((N_PAD, 64), jnp.float32),
    )(h, agg01,
      gp['mlp1']['W'], r2(gp['mlp1']['b']), r2(gp['mlp_ln']['g']),
      r2(gp['mlp_ln']['b']), gp['mlp2']['W'], r2(gp['mlp2']['b']),
      r2(lp['norm']['g']), r2(lp['norm']['b']),
      pp['lin']['W'], r2(pp['lin']['b']), r2(pp['ln']['g']),
      r2(pp['ln']['b']))


def _qkv_body(h_ref, qw, qb, kw, kb, vw, vb, sw, sb,
              q_ref, k_ref, v_ref, s_ref):
    h = h_ref[...]
    q_ref[...] = _dot(h, qw[...]) + qb[...]
    k_ref[...] = _dot(h, kw[...]) + kb[...]
    v_ref[...] = _dot(h, vw[...]) + vb[...]
    s_ref[...] = _dot(h, sw[...]) + sb[...]


def _qkv(h, tp):
    r2 = lambda v: v.reshape(1, -1)
    outs = pl.pallas_call(
        _qkv_body,
        grid=(N_PAD // _NB,),
        in_specs=[pl.BlockSpec((_NB, 64), _eblk)] + [
            _w_spec((64, 64)) if i % 2 == 0 else _w_spec((1, 64))
            for i in range(8)],
        out_specs=[pl.BlockSpec((_NB, 64), _eblk) for _ in range(4)],
        out_shape=[jax.ShapeDtypeStruct((N_PAD, 64), jnp.float32)] * 4,
    )(h, tp['q']['W'], r2(tp['q']['b']), tp['k']['W'], r2(tp['k']['b']),
      tp['v']['W'], r2(tp['v']['b']), tp['skip']['W'], r2(tp['skip']['b']))
    return outs


def _tc_edge_body(e_ref, qd_ref, kk_ref, vv_ref, w1, b1, g1, bb1, w2, b2,
                  eo_ref, a_ref, mb_ref):
    ep = _silu(_ln(_dot(e_ref[...], w1[...]) + b1[...], g1[...], bb1[...]))
    eo_ref[...] = ep
    ee = _dot(ep, w2[...]) + b2[...]
    kj = kk_ref[...] + ee
    t = qd_ref[...] * kj
    lanes = lax.broadcasted_iota(jnp.int32, (64, 8), 0)
    heads = lax.broadcasted_iota(jnp.int32, (64, 8), 1)
    sel = (lanes // 8 == heads).astype(jnp.float32)
    a8 = _dot(t, sel) * (1.0 / np.sqrt(8.0))
    a64 = _dot(a8, sel.T)
    a_ref[0] = a64[:, :32]
    a_ref[1] = a64[:, 32:]
    mb = vv_ref[...] + ee
    mb_ref[0] = mb[:, :32]
    mb_ref[1] = mb[:, 32:]


def _tc_edge(e, qd, kk, vv, lp):
    tp = lp['tc']
    r2 = lambda v: v.reshape(1, -1)
    return pl.pallas_call(
        _tc_edge_body,
        grid=(EP // _EB,),
        in_specs=_edge_specs(4) + [
            _w_spec((64, 64)), _w_spec((1, 64)), _w_spec((1, 64)),
            _w_spec((1, 64)), _w_spec((64, 64)), _w_spec((1, 64))],
        out_specs=[pl.BlockSpec((_EB, 64), _eblk), _split_spec(),
                   _split_spec()],
        out_shape=[jax.ShapeDtypeStruct((EP, 64), jnp.float32),
                   jax.ShapeDtypeStruct((2, EP, 32), jnp.float32),
                   jax.ShapeDtypeStruct((2, EP, 32), jnp.float32)],
    )(e, qd, kk, vv,
      lp['ec_lin']['W'], r2(lp['ec_lin']['b']),
      r2(lp['ec_ln']['g']), r2(lp['ec_ln']['b']),
      tp['e']['W'], r2(tp['e']['b']))


def _cmean_body(cs_ref, cnt_ref, out_ref):
    cs = jnp.concatenate([cs_ref[0], cs_ref[1]], axis=1)
    cnt = cnt_ref[0][:, :1]
    out_ref[...] = jnp.where(cnt > 0.0, cs / jnp.maximum(cnt, 1.0), 0.0)


def _cmean(csum01, cnt01):
    return pl.pallas_call(
        _cmean_body,
        grid=(N_PAD // _NB,),
        in_specs=[_nsplit_spec(), _nsplit_spec()],
        out_specs=pl.BlockSpec((_NB, 64), _eblk),
        out_shape=jax.ShapeDtypeStruct((N_PAD, 64), jnp.float32),
    )(csum01, cnt01)


def _ex_body(a_ref, cd_ref, ex_ref):
    a = jnp.concatenate([a_ref[0], a_ref[1]], axis=1)
    ex = jnp.exp(a - cd_ref[...])
    ex_ref[0] = ex[:, :32]
    ex_ref[1] = ex[:, 32:]


def _ex(a01, cd):
    return pl.pallas_call(
        _ex_body,
        grid=(EP // _EB,),
        in_specs=[_split_spec(), pl.BlockSpec((_EB, 64), _eblk)],
        out_specs=_split_spec(),
        out_shape=jax.ShapeDtypeStruct((2, EP, 32), jnp.float32),
    )(a01, cd)


def _msg_body(ex_ref, mb_ref, dd_ref, msg_ref):
    ex = jnp.concatenate([ex_ref[0], ex_ref[1]], axis=1)
    mb = jnp.concatenate([mb_ref[0], mb_ref[1]], axis=1)
    a = ex / (dd_ref[...] + 1e-16)
    m = mb * a
    msg_ref[0] = m[:, :32]
    msg_ref[1] = m[:, 32:]


def _msg(ex01, mb01, dd):
    return pl.pallas_call(
        _msg_body,
        grid=(EP // _EB,),
        in_specs=[_split_spec(), _split_spec(),
                  pl.BlockSpec((_EB, 64), _eblk)],
        out_specs=_split_spec(),
        out_shape=jax.ShapeDtypeStruct((2, EP, 32), jnp.float32),
    )(ex01, mb01, dd)


def _gate_body(h_ref, out01_ref, xr_ref, wb, ng, nb, o_ref):
    h = h_ref[...]
    out = jnp.concatenate([out01_ref[0], out01_ref[1]], axis=1)
    xr = xr_ref[...]
    cat = jnp.concatenate([out, xr, out - xr], axis=1)
    beta = jax.nn.sigmoid(_dot(cat, wb[...]))
    hn = beta * xr + (1.0 - beta) * out
    hn = _ln(hn, ng[...], nb[...])
    o_ref[...] = h + hn


def _gate(h, out01, xr, lp):
    r2 = lambda v: v.reshape(1, -1)
    return pl.pallas_call(
        _gate_body,
        grid=(N_PAD // _NB,),
        in_specs=[pl.BlockSpec((_NB, 64), _eblk), _nsplit_spec(),
                  pl.BlockSpec((_NB, 64), _eblk),
                  _w_spec((192, 1)), _w_spec((1, 64)), _w_spec((1, 64))],
        out_specs=pl.BlockSpec((_NB, 64), _eblk),
        out_shape=jax.ShapeDtypeStruct((N_PAD, 64), jnp.float32),
    )(h, out01, xr, lp['tc']['Wbeta'],
      r2(lp['norm']['g']), r2(lp['norm']['b']))


_S2S_CHUNK = 2944
_S2S_NCHUNK = N_PAD // _S2S_CHUNK


def _s2s_head_body(*refs):
    (h_ref, oh_ref, gf_ref,
     wih0, whh0, bih0, bhh0,
     wih1, whh1, bih1, bhh1,
     rp_w, rp_b, rp_g, rp_b2,
     gm_w1, gm_b1, gm_g, gm_bln, gm_w2, gm_b2,
     fm_w1, fm_b1, fm_g1, fm_bln1,
     fm_w2, fm_b2, fm_g2, fm_bln2,
     fm_w3, fm_b3,
     out_ref) = refs

    B = NUM_GRAPHS
    H = HID
    CH = _S2S_CHUNK

    def one_s2s(wih, whh, bih, bhh):
        hst = jnp.zeros((B, H), jnp.float32)
        cst = jnp.zeros((B, H), jnp.float32)
        qs = jnp.zeros((B, 2 * H), jnp.float32)
        for _ in range(4):
            z = (_dot(qs, wih[...]) + bih[...]
                 + _dot(hst, whh[...]) + bhh[...])
            i_g, f_g, g_g, o_g = jnp.split(z, 4, axis=-1)
            cst = jax.nn.sigmoid(f_g) * cst + jax.nn.sigmoid(i_g) * jnp.tanh(g_g)
            hst = jax.nn.sigmoid(o_g) * jnp.tanh(cst)

            def max_step(c, emax):
                xc = h_ref[pl.ds(c * CH, CH), :]
                mc = oh_ref[:, pl.ds(c * CH, CH)]
                e0 = lax.dot_general(hst, xc, (((1,), (1,)), ((), ())),
                                     preferred_element_type=jnp.float32)
                emsk = jnp.where(mc > 0.0, e0, -jnp.inf)
                return jnp.maximum(emax, jnp.max(emsk, axis=1, keepdims=True))

            emax = lax.fori_loop(0, _S2S_NCHUNK, max_step,
                                 jnp.full((B, 1), -jnp.inf, jnp.float32))
            emax = jnp.where(jnp.isfinite(emax), emax, 0.0)

            def acc_step(c, carry):
                den, num = carry
                xc = h_ref[pl.ds(c * CH, CH), :]
                mc = oh_ref[:, pl.ds(c * CH, CH)]
                e0 = lax.dot_general(hst, xc, (((1,), (1,)), ((), ())),
                                     preferred_element_type=jnp.float32)
                eng = jnp.sum(e0 * mc, axis=0, keepdims=True)
                esel = jnp.sum(mc * emax, axis=0, keepdims=True)
                ex = jnp.exp(eng - esel)
                w = mc * ex
                den = den + jnp.sum(w, axis=1, keepdims=True)
                num = num + _dot(w, xc)
                return den, num

            den, num = lax.fori_loop(
                0, _S2S_NCHUNK, acc_step,
                (jnp.zeros((B, 1), jnp.float32),
                 jnp.zeros((B, H), jnp.float32)))
            r = num / (den + 1e-16)
            qs = jnp.concatenate([hst, r], axis=1)
        return qs

    qs0 = one_s2s(wih0, whh0, bih0, bhh0)
    qs1 = one_s2s(wih1, whh1, bih1, bhh1)
    hr = jnp.concatenate([qs0, qs1], axis=1)
    hr = _silu(_ln(_dot(hr, rp_w[...]) + rp_b[...], rp_g[...], rp_b2[...]))
    g = _silu(_ln(_dot(gf_ref[...], gm_w1[...]) + gm_b1[...],
                  gm_g[...], gm_bln[...]))
    g = _dot(g, gm_w2[...]) + gm_b2[...]
    comb = jnp.concatenate([hr, g], axis=1)
    f = _silu(_ln(_dot(comb, fm_w1[...]) + fm_b1[...], fm_g1[...],
                  fm_bln1[...]))
    f = _silu(_ln(_dot(f, fm_w2[...]) + fm_b2[...], fm_g2[...],
                  fm_bln2[...]))
    f = _dot(f, fm_w3[...]) + fm_b3[...]
    out_ref[...] = _softplus(f).T


def _s2s_head(h_pad, oh, gf, params):
    s2s = params['s2s']
    rp = params['rp']
    gm = params['gm']
    fm = params['fm']
    r2 = lambda v: v.reshape(1, -1)
    args = [
        h_pad, oh, gf,
        s2s[0]['Wih'], s2s[0]['Whh'], r2(s2s[0]['bih']), r2(s2s[0]['bhh']),
        s2s[1]['Wih'], s2s[1]['Whh'], r2(s2s[1]['bih']), r2(s2s[1]['bhh']),
        rp['lin']['W'], r2(rp['lin']['b']), r2(rp['ln']['g']), r2(rp['ln']['b']),
        gm['lin1']['W'], r2(gm['lin1']['b']), r2(gm['ln']['g']), r2(gm['ln']['b']),
        gm['lin2']['W'], r2(gm['lin2']['b']),
        fm['lin1']['W'], r2(fm['lin1']['b']), r2(fm['ln1']['g']), r2(fm['ln1']['b']),
        fm['lin2']['W'], r2(fm['lin2']['b']), r2(fm['ln2']['g']), r2(fm['ln2']['b']),
        fm['lin3']['W'], r2(fm['lin3']['b']),
    ]
    out = pl.pallas_call(
        _s2s_head_body,
        out_shape=jax.ShapeDtypeStruct((1, NUM_GRAPHS), jnp.float32),
    )(*args)
    return out.reshape(NUM_GRAPHS)


def kernel(x, edge_attr, nA, nB, system_size, params, edge_index, batch):
    src = edge_index[0]
    dst = edge_index[1]
    epad = EP - N_EDGES
    src3g = jnp.pad(src, (0, epad)).reshape(32, _OPS_W, 128)
    dst_p = jnp.pad(dst, (0, epad), constant_values=N_NODES)
    dst3g = dst_p.reshape(32, _OPS_W, 128)
    dst3s = dst_p.reshape(16, _OPS_T, 128)

    x_pad = jnp.pad(x, ((0, N_PAD - N_NODES), (0, 0)))
    ea_pad = jnp.pad(edge_attr, ((0, epad), (0, 0)))
    zeros32 = jnp.zeros((N_PAD, 32), jnp.float32)
    ones01 = jnp.ones((2, EP, 32), jnp.float32)

    h = _encode_nodes(x_pad, params['node_enc'])
    e = _encode_edges(ea_pad, params['edge_enc'])

    cnt01 = _sc_scatter_add(ones01, dst3s, zeros32)

    for i in range(NUM_LAYERS):
        lp = params['layers'][i]
        if i % 2 == 0:
            hs = _sc_gather(h, src3g)
            e, msg01 = _gine_edge(e, hs, lp)
            agg01 = _sc_scatter_add(msg01, dst3s, zeros32)
            h = _gine_node(h, agg01, lp, params['pool'][i // 2])
        else:
            q, kk, vv, xr = _qkv(h, lp['tc'])
            qd = _sc_gather(q, dst3g)
            ks = _sc_gather(kk, src3g)
            vs = _sc_gather(vv, src3g)
            e, a01, mb01 = _tc_edge(e, qd, ks, vs, lp)
            csum01 = _sc_scatter_add(a01, dst3s, zeros32)
            c = _cmean(csum01, cnt01)
            cd = _sc_gather(c, dst3g)
            ex01 = _ex(a01, cd)
            den01 = _sc_scatter_add(ex01, dst3s, zeros32)
            den = jnp.concatenate([den01[0], den01[1]], axis=1)
            dd = _sc_gather(den, dst3g)
            msg01 = _msg(ex01, mb01, dd)
            out01 = _sc_scatter_add(msg01, dst3s, zeros32)
            h = _gate(h, out01, xr, lp)

    gids = jnp.arange(NUM_GRAPHS, dtype=jnp.int32)
    batch_pad = jnp.pad(batch, (0, N_PAD - N_NODES), constant_values=-1)
    oh = (batch_pad[None, :] == gids[:, None]).astype(jnp.float32)
    nAn = nA[:, 0] / (system_size[:, 0] + 1e-10)
    nBn = nB[:, 0] / (system_size[:, 0] + 1e-10)
    gf = jnp.stack([nAn, nBn], axis=1)
    return _s2s_head(h, oh, gf, params)


# speedup vs baseline: 15.9726x; 1.1150x over previous
import functools

import jax
import jax.numpy as jnp
import numpy as np
from jax import lax
from jax.experimental import pallas as pl
from jax.experimental.pallas import tpu as pltpu
from jax.experimental.pallas import tpu_sc as plsc

N_NODES = 50000
N_PAD = 50048
N_EDGES = 800000
EP = 802816
HID = 64
NUM_LAYERS = 10
NUM_GRAPHS = 16

_EB = 2048
_NB = 3128

_mesh_cache = []


def _mesh():
    if not _mesh_cache:
        _mesh_cache.append(plsc.VectorSubcoreMesh(
            core_axis_name="c", subcore_axis_name="s"))
    return _mesh_cache[0]


def _ln(x, g, b, eps=1e-5):
    m = jnp.mean(x, axis=-1, keepdims=True)
    v = jnp.mean((x - m) ** 2, axis=-1, keepdims=True)
    return (x - m) / jnp.sqrt(v + eps) * g + b


def _silu(x):
    return x * jax.nn.sigmoid(x)


def _softplus(x):
    return jnp.maximum(x, 0.0) + jnp.log1p(jnp.exp(-jnp.abs(x)))


def _dot(a, b):
    return jnp.dot(a, b, preferred_element_type=jnp.float32)


_OPS_W = EP // 128 // 32
_OPS_T = EP // 128 // 16
_TROWS = N_PAD // 16


def _sc_gather(table, idx2d):
    @functools.partial(
        pl.kernel, mesh=_mesh(),
        compiler_params=pltpu.CompilerParams(use_tc_tiling_on_sc=False),
        out_type=jax.ShapeDtypeStruct((EP, 64), jnp.float32),
        scratch_types=[
            pltpu.VMEM((_OPS_W, 128), jnp.int32),
            pltpu.VMEM((512, 64), jnp.float32),
            pltpu.VMEM((512, 64), jnp.float32),
            pltpu.SemaphoreType.DMA,
            pltpu.SemaphoreType.DMA,
            pltpu.SemaphoreType.DMA,
            pltpu.SemaphoreType.DMA,
        ],
    )
    def k(table_hbm, idx_hbm, out_hbm, idx_v, st0, st1, g0, g1, w0, w1):
        cid = lax.axis_index("c")
        sid = lax.axis_index("s")
        wid = sid * 2 + cid
        pltpu.sync_copy(idx_hbm.at[wid], idx_v)
        ebase = wid * _OPS_W * 128

        def fire4(g, stage, sem):
            for b in range(4):
                pltpu.async_copy(table_hbm.at[idx_v.at[g * 4 + b]],
                                 stage.at[pl.ds(b * 128, 128), :], sem)

        def wait4(stage, sem):
            pltpu.make_async_copy(table_hbm.at[idx_v.at[0]], stage, sem).wait()

        def wr(g, stage, sem):
            pltpu.async_copy(stage,
                             out_hbm.at[pl.ds(ebase + g * 512, 512), :], sem)

        def wrwait(stage, sem):
            pltpu.make_async_copy(
                stage, out_hbm.at[pl.ds(ebase, 512), :], sem).wait()

        fire4(0, st0, g0)

        def body(i, carry):
            fire4(2 * i + 1, st1, g1)
            wait4(st0, g0)
            wr(2 * i, st0, w0)
            wrwait(st0, w0)
            fire4(2 * i + 2, st0, g0)
            wait4(st1, g1)
            wr(2 * i + 1, st1, w1)
            wrwait(st1, w1)
            return carry

        lax.fori_loop(0, 24, body, 0)
        wait4(st0, g0)
        pltpu.sync_copy(st0, out_hbm.at[pl.ds(ebase + 48 * 512, 512), :])

    return k(table, idx2d)


def _sc_scatter_add(msg01, dst3s, zeros32):
    @functools.partial(
        pl.kernel, mesh=_mesh(),
        compiler_params=pltpu.CompilerParams(use_tc_tiling_on_sc=False),
        out_type=jax.ShapeDtypeStruct((2, N_PAD, 32), jnp.float32),
        scratch_types=[
            pltpu.VMEM((56, 128), jnp.int32),
            pltpu.VMEM((128, 32), jnp.float32),
            pltpu.VMEM((128, 32), jnp.float32),
            pltpu.VMEM((128, 32), jnp.float32),
            pltpu.VMEM((128, 32), jnp.float32),
            pltpu.VMEM_SHARED((N_PAD, 32), jnp.float32),
            pltpu.SemaphoreType.DMA,
            pltpu.SemaphoreType.DMA,
            pltpu.SemaphoreType.DMA,
            pltpu.SemaphoreType.DMA,
            pltpu.SemaphoreType.DMA,
            pltpu.SemaphoreType.DMA,
            pltpu.SemaphoreType.DMA,
            pltpu.SemaphoreType.DMA,
        ],
    )
    def k(msg_hbm, idx_hbm, z_hbm, out_hbm, idx_v, b0, b1, b2, b3, acc,
          l0, l1, l2, l3, s0, s1, s2, s3):
        cid = lax.axis_index("c")
        sid = lax.axis_index("s")
        bufs = (b0, b1, b2, b3)
        lsems = (l0, l1, l2, l3)
        ssems = (s0, s1, s2, s3)
        tbase = sid * _OPS_T * 128

        @pl.when(sid == 0)
        def _():
            pltpu.sync_copy(z_hbm, acc)

        plsc.subcore_barrier()

        def load(op, b, sem):
            pltpu.async_copy(
                msg_hbm.at[cid, pl.ds(tbase + op * 128, 128), :], bufs[b],
                sem)

        def loadwait(b, sem):
            pltpu.make_async_copy(
                msg_hbm.at[cid, pl.ds(tbase, 128), :], bufs[b], sem).wait()

        for b in range(4):
            load(b, b, lsems[b])

        def outer(m, carry):
            pltpu.sync_copy(idx_hbm.at[sid, pl.ds(m * 56, 56), :], idx_v)

            def inner(g, carry2):
                for b in range(4):
                    opl = g * 4 + b
                    op = m * 56 + opl
                    loadwait(b, lsems[b])
                    pltpu.async_copy(bufs[b], acc.at[idx_v.at[opl]],
                                     ssems[b], add=True)
                for b in range(4):
                    opn = m * 56 + g * 4 + b + 4
                    pltpu.make_async_copy(
                        bufs[b], acc.at[idx_v.at[0]], ssems[b]).wait()

                    @pl.when(opn < _OPS_T)
                    def _():
                        load(opn, b, lsems[b])

                return carry2

            return lax.fori_loop(0, 14, inner, carry)

        lax.fori_loop(0, 7, outer, 0)
        plsc.subcore_barrier()
        pltpu.sync_copy(acc.at[pl.ds(sid * _TROWS, _TROWS), :],
                        out_hbm.at[cid, pl.ds(sid * _TROWS, _TROWS), :])

    return k(msg01, dst3s, zeros32)


def _eblk(i):
    return (i, 0)


def _edge_specs(n):
    return [pl.BlockSpec((_EB, 64), _eblk) for _ in range(n)]


def _w_spec(shape):
    return pl.BlockSpec(shape, lambda i: tuple(0 for _ in shape))


def _split_spec():
    return pl.BlockSpec((2, _EB, 32), lambda i: (0, i, 0))


def _nsplit_spec():
    return pl.BlockSpec((2, _NB, 32), lambda i: (0, i, 0))


def _enc_body(x_ref, w, b, g, bb, out_ref):
    z = _ln(_dot(x_ref[...], w[...]) + b[...], g[...], bb[...])
    out_ref[...] = _silu(z)


def _encode_nodes(x_pad, p):
    return pl.pallas_call(
        _enc_body,
        grid=(N_PAD // _NB,),
        in_specs=[pl.BlockSpec((_NB, 4), _eblk), _w_spec((4, HID)),
                  _w_spec((1, HID)), _w_spec((1, HID)), _w_spec((1, HID))],
        out_specs=pl.BlockSpec((_NB, 64), _eblk),
        out_shape=jax.ShapeDtypeStruct((N_PAD, 64), jnp.float32),
    )(x_pad, p['lin']['W'], p['lin']['b'].reshape(1, -1),
      p['ln']['g'].reshape(1, -1), p['ln']['b'].reshape(1, -1))


def _encode_edges(ea_pad, p):
    return pl.pallas_call(
        _enc_body,
        grid=(EP // _EB,),
        in_specs=[pl.BlockSpec((_EB, 3), _eblk), _w_spec((3, HID)),
                  _w_spec((1, HID)), _w_spec((1, HID)), _w_spec((1, HID))],
        out_specs=pl.BlockSpec((_EB, 64), _eblk),
        out_shape=jax.ShapeDtypeStruct((EP, 64), jnp.float32),
    )(ea_pad, p['lin']['W'], p['lin']['b'].reshape(1, -1),
      p['ln']['g'].reshape(1, -1), p['ln']['b'].reshape(1, -1))


def _gine_edge_body(e_ref, hs_ref, w1, b1, g1, bb1, w2, b2,
                    eo_ref, msg_ref):
    ep = _silu(_ln(_dot(e_ref[...], w1[...]) + b1[...], g1[...], bb1[...]))
    eo_ref[...] = ep
    ee = _dot(ep, w2[...]) + b2[...]
    m = jnp.maximum(hs_ref[...] + ee, 0.0)
    msg_ref[0] = m[:, :32]
    msg_ref[1] = m[:, 32:]


def _gine_edge(e, hs, lp):
    gp = lp['gine']
    return pl.pallas_call(
        _gine_edge_body,
        grid=(EP // _EB,),
        in_specs=_edge_specs(2) + [
            _w_spec((64, 64)), _w_spec((1, 64)), _w_spec((1, 64)),
            _w_spec((1, 64)), _w_spec((64, 64)), _w_spec((1, 64))],
        out_specs=[pl.BlockSpec((_EB, 64), _eblk), _split_spec()],
        out_shape=[jax.ShapeDtypeStruct((EP, 64), jnp.float32),
                   jax.ShapeDtypeStruct((2, EP, 32), jnp.float32)],
    )(e, hs,
      lp['ec_lin']['W'], lp['ec_lin']['b'].reshape(1, -1),
      lp['ec_ln']['g'].reshape(1, -1), lp['ec_ln']['b'].reshape(1, -1),
      gp['lin_edge']['W'], gp['lin_edge']['b'].reshape(1, -1))


def _gine_node_body(h_ref, agg_ref, m1w, m1b, mlg, mlb, m2w, m2b,
                    ng, nb, pw, pb, pg, pbb, out_ref):
    h = h_ref[...]
    agg = jnp.concatenate([agg_ref[0], agg_ref[1]], axis=1)
    z = h + agg
    z = _silu(_ln(_dot(z, m1w[...]) + m1b[...], mlg[...], mlb[...]))
    z = _dot(z, m2w[...]) + m2b[...]
    hn = _ln(z, ng[...], nb[...])
    h2 = h + hn
    h2 = _silu(_ln(_dot(h2, pw[...]) + pb[...], pg[...], pbb[...]))
    out_ref[...] = h2


def _gine_node(h, agg01, lp, pp):
    gp = lp['gine']
    r2 = lambda v: v.reshape(1, -1)
    return pl.pallas_call(
        _gine_node_body,
        grid=(N_PAD // _NB,),
        in_specs=[pl.BlockSpec((_NB, 64), _eblk), _nsplit_spec(),
                  _w_spec((64, 64)), _w_spec((1, 64)), _w_spec((1, 64)),
                  _w_spec((1, 64)), _w_spec((64, 64)), _w_spec((1, 64)),
                  _w_spec((1, 64)), _w_spec((1, 64)),
                  _w_spec((64, 64)), _w_spec((1, 64)), _w_spec((1, 64)),
                  _w_spec((1, 64))],
        out_specs=pl.BlockSpec((_NB, 64), _eblk),
        out_shape=jax.ShapeDtypeStruct((N_PAD, 64), jnp.float32),
    )(h, agg01,
      gp['mlp1']['W'], r2(gp['mlp1']['b']), r2(gp['mlp_ln']['g']),
      r2(gp['mlp_ln']['b']), gp['mlp2']['W'], r2(gp['mlp2']['b']),
      r2(lp['norm']['g']), r2(lp['norm']['b']),
      pp['lin']['W'], r2(pp['lin']['b']), r2(pp['ln']['g']),
      r2(pp['ln']['b']))


def _qkv_body(h_ref, qw, qb, kw, kb, vw, vb, sw, sb,
              q_ref, k_ref, v_ref, s_ref):
    h = h_ref[...]
    q_ref[...] = _dot(h, qw[...]) + qb[...]
    k_ref[...] = _dot(h, kw[...]) + kb[...]
    v_ref[...] = _dot(h, vw[...]) + vb[...]
    s_ref[...] = _dot(h, sw[...]) + sb[...]


def _qkv(h, tp):
    r2 = lambda v: v.reshape(1, -1)
    outs = pl.pallas_call(
        _qkv_body,
        grid=(N_PAD // _NB,),
        in_specs=[pl.BlockSpec((_NB, 64), _eblk)] + [
            _w_spec((64, 64)) if i % 2 == 0 else _w_spec((1, 64))
            for i in range(8)],
        out_specs=[pl.BlockSpec((_NB, 64), _eblk) for _ in range(4)],
        out_shape=[jax.ShapeDtypeStruct((N_PAD, 64), jnp.float32)] * 4,
    )(h, tp['q']['W'], r2(tp['q']['b']), tp['k']['W'], r2(tp['k']['b']),
      tp['v']['W'], r2(tp['v']['b']), tp['skip']['W'], r2(tp['skip']['b']))
    return outs


def _tc_edge_body(e_ref, qd_ref, kk_ref, vv_ref, w1, b1, g1, bb1, w2, b2,
                  eo_ref, a_ref, mb_ref):
    ep = _silu(_ln(_dot(e_ref[...], w1[...]) + b1[...], g1[...], bb1[...]))
    eo_ref[...] = ep
    ee = _dot(ep, w2[...]) + b2[...]
    kj = kk_ref[...] + ee
    t = qd_ref[...] * kj
    lanes = lax.broadcasted_iota(jnp.int32, (64, 8), 0)
    heads = lax.broadcasted_iota(jnp.int32, (64, 8), 1)
    sel = (lanes // 8 == heads).astype(jnp.float32)
    a8 = _dot(t, sel) * (1.0 / np.sqrt(8.0))
    a64 = _dot(a8, sel.T)
    a_ref[0] = a64[:, :32]
    a_ref[1] = a64[:, 32:]
    mb = vv_ref[...] + ee
    mb_ref[0] = mb[:, :32]
    mb_ref[1] = mb[:, 32:]


def _tc_edge(e, qd, kk, vv, lp):
    tp = lp['tc']
    r2 = lambda v: v.reshape(1, -1)
    return pl.pallas_call(
        _tc_edge_body,
        grid=(EP // _EB,),
        in_specs=_edge_specs(4) + [
            _w_spec((64, 64)), _w_spec((1, 64)), _w_spec((1, 64)),
            _w_spec((1, 64)), _w_spec((64, 64)), _w_spec((1, 64))],
        out_specs=[pl.BlockSpec((_EB, 64), _eblk), _split_spec(),
                   _split_spec()],
        out_shape=[jax.ShapeDtypeStruct((EP, 64), jnp.float32),
                   jax.ShapeDtypeStruct((2, EP, 32), jnp.float32),
                   jax.ShapeDtypeStruct((2, EP, 32), jnp.float32)],
    )(e, qd, kk, vv,
      lp['ec_lin']['W'], r2(lp['ec_lin']['b']),
      r2(lp['ec_ln']['g']), r2(lp['ec_ln']['b']),
      tp['e']['W'], r2(tp['e']['b']))


def _cmean_body(cs_ref, cnt_ref, out_ref):
    cs = jnp.concatenate([cs_ref[0], cs_ref[1]], axis=1)
    cnt = cnt_ref[0][:, :1]
    out_ref[...] = jnp.where(cnt > 0.0, cs / jnp.maximum(cnt, 1.0), 0.0)


def _cmean(csum01, cnt01):
    return pl.pallas_call(
        _cmean_body,
        grid=(N_PAD // _NB,),
        in_specs=[_nsplit_spec(), _nsplit_spec()],
        out_specs=pl.BlockSpec((_NB, 64), _eblk),
        out_shape=jax.ShapeDtypeStruct((N_PAD, 64), jnp.float32),
    )(csum01, cnt01)


def _ex_body(a_ref, cd_ref, ex_ref):
    a = jnp.concatenate([a_ref[0], a_ref[1]], axis=1)
    ex = jnp.exp(a - cd_ref[...])
    ex_ref[0] = ex[:, :32]
    ex_ref[1] = ex[:, 32:]


def _ex(a01, cd):
    return pl.pallas_call(
        _ex_body,
        grid=(EP // _EB,),
        in_specs=[_split_spec(), pl.BlockSpec((_EB, 64), _eblk)],
        out_specs=_split_spec(),
        out_shape=jax.ShapeDtypeStruct((2, EP, 32), jnp.float32),
    )(a01, cd)


def _msg_body(ex_ref, mb_ref, dd_ref, msg_ref):
    ex = jnp.concatenate([ex_ref[0], ex_ref[1]], axis=1)
    mb = jnp.concatenate([mb_ref[0], mb_ref[1]], axis=1)
    a = ex / (dd_ref[...] + 1e-16)
    m = mb * a
    msg_ref[0] = m[:, :32]
    msg_ref[1] = m[:, 32:]


def _msg(ex01, mb01, dd):
    return pl.pallas_call(
        _msg_body,
        grid=(EP // _EB,),
        in_specs=[_split_spec(), _split_spec(),
                  pl.BlockSpec((_EB, 64), _eblk)],
        out_specs=_split_spec(),
        out_shape=jax.ShapeDtypeStruct((2, EP, 32), jnp.float32),
    )(ex01, mb01, dd)


def _gate_body(h_ref, out01_ref, xr_ref, wb, ng, nb, o_ref):
    h = h_ref[...]
    out = jnp.concatenate([out01_ref[0], out01_ref[1]], axis=1)
    xr = xr_ref[...]
    cat = jnp.concatenate([out, xr, out - xr], axis=1)
    beta = jax.nn.sigmoid(_dot(cat, wb[...]))
    hn = beta * xr + (1.0 - beta) * out
    hn = _ln(hn, ng[...], nb[...])
    o_ref[...] = h + hn


def _gate(h, out01, xr, lp):
    r2 = lambda v: v.reshape(1, -1)
    return pl.pallas_call(
        _gate_body,
        grid=(N_PAD // _NB,),
        in_specs=[pl.BlockSpec((_NB, 64), _eblk), _nsplit_spec(),
                  pl.BlockSpec((_NB, 64), _eblk),
                  _w_spec((192, 1)), _w_spec((1, 64)), _w_spec((1, 64))],
        out_specs=pl.BlockSpec((_NB, 64), _eblk),
        out_shape=jax.ShapeDtypeStruct((N_PAD, 64), jnp.float32),
    )(h, out01, xr, lp['tc']['Wbeta'],
      r2(lp['norm']['g']), r2(lp['norm']['b']))


_S2S_CHUNK = 2944
_S2S_NCHUNK = N_PAD // _S2S_CHUNK


def _s2s_head_body(*refs):
    (h_ref, oh_ref, gf_ref,
     wih0, whh0, bih0, bhh0,
     wih1, whh1, bih1, bhh1,
     rp_w, rp_b, rp_g, rp_b2,
     gm_w1, gm_b1, gm_g, gm_bln, gm_w2, gm_b2,
     fm_w1, fm_b1, fm_g1, fm_bln1,
     fm_w2, fm_b2, fm_g2, fm_bln2,
     fm_w3, fm_b3,
     out_ref) = refs

    B = NUM_GRAPHS
    H = HID
    CH = _S2S_CHUNK

    def one_s2s(wih, whh, bih, bhh):
        hst = jnp.zeros((B, H), jnp.float32)
        cst = jnp.zeros((B, H), jnp.float32)
        qs = jnp.zeros((B, 2 * H), jnp.float32)
        for _ in range(4):
            z = (_dot(qs, wih[...]) + bih[...]
                 + _dot(hst, whh[...]) + bhh[...])
            i_g, f_g, g_g, o_g = jnp.split(z, 4, axis=-1)
            cst = jax.nn.sigmoid(f_g) * cst + jax.nn.sigmoid(i_g) * jnp.tanh(g_g)
            hst = jax.nn.sigmoid(o_g) * jnp.tanh(cst)

            def max_step(c, emax):
                xc = h_ref[pl.ds(c * CH, CH), :]
                mc = oh_ref[:, pl.ds(c * CH, CH)]
                e0 = lax.dot_general(hst, xc, (((1,), (1,)), ((), ())),
                                     preferred_element_type=jnp.float32)
                emsk = jnp.where(mc > 0.0, e0, -jnp.inf)
                return jnp.maximum(emax, jnp.max(emsk, axis=1, keepdims=True))

            emax = lax.fori_loop(0, _S2S_NCHUNK, max_step,
                                 jnp.full((B, 1), -jnp.inf, jnp.float32))
            emax = jnp.where(jnp.isfinite(emax), emax, 0.0)

            def acc_step(c, carry):
                den, num = carry
                xc = h_ref[pl.ds(c * CH, CH), :]
                mc = oh_ref[:, pl.ds(c * CH, CH)]
                e0 = lax.dot_general(hst, xc, (((1,), (1,)), ((), ())),
                                     preferred_element_type=jnp.float32)
                eng = jnp.sum(e0 * mc, axis=0, keepdims=True)
                esel = jnp.sum(mc * emax, axis=0, keepdims=True)
                ex = jnp.exp(eng - esel)
                w = mc * ex
                den = den + jnp.sum(w, axis=1, keepdims=True)
                num = num + _dot(w, xc)
                return den, num

            den, num = lax.fori_loop(
                0, _S2S_NCHUNK, acc_step,
                (jnp.zeros((B, 1), jnp.float32),
                 jnp.zeros((B, H), jnp.float32)))
            r = num / (den + 1e-16)
            qs = jnp.concatenate([hst, r], axis=1)
        return qs

    qs0 = one_s2s(wih0, whh0, bih0, bhh0)
    qs1 = one_s2s(wih1, whh1, bih1, bhh1)
    hr = jnp.concatenate([qs0, qs1], axis=1)
    hr = _silu(_ln(_dot(hr, rp_w[...]) + rp_b[...], rp_g[...], rp_b2[...]))
    g = _silu(_ln(_dot(gf_ref[...], gm_w1[...]) + gm_b1[...],
                  gm_g[...], gm_bln[...]))
    g = _dot(g, gm_w2[...]) + gm_b2[...]
    comb = jnp.concatenate([hr, g], axis=1)
    f = _silu(_ln(_dot(comb, fm_w1[...]) + fm_b1[...], fm_g1[...],
                  fm_bln1[...]))
    f = _silu(_ln(_dot(f, fm_w2[...]) + fm_b2[...], fm_g2[...],
                  fm_bln2[...]))
    f = _dot(f, fm_w3[...]) + fm_b3[...]
    out_ref[...] = _softplus(f).T


def _s2s_head(h_pad, oh, gf, params):
    s2s = params['s2s']
    rp = params['rp']
    gm = params['gm']
    fm = params['fm']
    r2 = lambda v: v.reshape(1, -1)
    args = [
        h_pad, oh, gf,
        s2s[0]['Wih'], s2s[0]['Whh'], r2(s2s[0]['bih']), r2(s2s[0]['bhh']),
        s2s[1]['Wih'], s2s[1]['Whh'], r2(s2s[1]['bih']), r2(s2s[1]['bhh']),
        rp['lin']['W'], r2(rp['lin']['b']), r2(rp['ln']['g']), r2(rp['ln']['b']),
        gm['lin1']['W'], r2(gm['lin1']['b']), r2(gm['ln']['g']), r2(gm['ln']['b']),
        gm['lin2']['W'], r2(gm['lin2']['b']),
        fm['lin1']['W'], r2(fm['lin1']['b']), r2(fm['ln1']['g']), r2(fm['ln1']['b']),
        fm['lin2']['W'], r2(fm['lin2']['b']), r2(fm['ln2']['g']), r2(fm['ln2']['b']),
        fm['lin3']['W'], r2(fm['lin3']['b']),
    ]
    out = pl.pallas_call(
        _s2s_head_body,
        out_shape=jax.ShapeDtypeStruct((1, NUM_GRAPHS), jnp.float32),
    )(*args)
    return out.reshape(NUM_GRAPHS)


def kernel(x, edge_attr, nA, nB, system_size, params, edge_index, batch):
    src = edge_index[0]
    dst = edge_index[1]
    epad = EP - N_EDGES
    src3g = jnp.pad(src, (0, epad)).reshape(32, _OPS_W, 128)
    dst_p = jnp.pad(dst, (0, epad), constant_values=N_NODES)
    dst3g = dst_p.reshape(32, _OPS_W, 128)
    dst3s = dst_p.reshape(16, _OPS_T, 128)

    x_pad = jnp.pad(x, ((0, N_PAD - N_NODES), (0, 0)))
    ea_pad = jnp.pad(edge_attr, ((0, epad), (0, 0)))
    zeros32 = jnp.zeros((N_PAD, 32), jnp.float32)
    ones01 = jnp.ones((2, EP, 32), jnp.float32)

    h = _encode_nodes(x_pad, params['node_enc'])
    e = _encode_edges(ea_pad, params['edge_enc'])

    cnt01 = _sc_scatter_add(ones01, dst3s, zeros32)

    for i in range(NUM_LAYERS):
        lp = params['layers'][i]
        if i % 2 == 0:
            hs = _sc_gather(h, src3g)
            e, msg01 = _gine_edge(e, hs, lp)
            agg01 = _sc_scatter_add(msg01, dst3s, zeros32)
            h = _gine_node(h, agg01, lp, params['pool'][i // 2])
        else:
            q, kk, vv, xr = _qkv(h, lp['tc'])
            qd = _sc_gather(q, dst3g)
            ks = _sc_gather(kk, src3g)
            vs = _sc_gather(vv, src3g)
            e, a01, mb01 = _tc_edge(e, qd, ks, vs, lp)
            csum01 = _sc_scatter_add(a01, dst3s, zeros32)
            c = _cmean(csum01, cnt01)
            cd = _sc_gather(c, dst3g)
            ex01 = _ex(a01, cd)
            den01 = _sc_scatter_add(ex01, dst3s, zeros32)
            den = jnp.concatenate([den01[0], den01[1]], axis=1)
            dd = _sc_gather(den, dst3g)
            msg01 = _msg(ex01, mb01, dd)
            out01 = _sc_scatter_add(msg01, dst3s, zeros32)
            h = _gate(h, out01, xr, lp)

    gids = jnp.arange(NUM_GRAPHS, dtype=jnp.int32)
    batch_pad = jnp.pad(batch, (0, N_PAD - N_NODES), constant_values=-1)
    oh = (batch_pad[None, :] == gids[:, None]).astype(jnp.float32)
    nAn = nA[:, 0] / (system_size[:, 0] + 1e-10)
    nBn = nB[:, 0] / (system_size[:, 0] + 1e-10)
    gf = jnp.stack([nAn, nBn], axis=1)
    return _s2s_head(h, oh, gf, params)


# speedup vs baseline: 15.9840x; 1.0007x over previous
import functools

import jax
import jax.numpy as jnp
import numpy as np
from jax import lax
from jax.experimental import pallas as pl
from jax.experimental.pallas import tpu as pltpu
from jax.experimental.pallas import tpu_sc as plsc

N_NODES = 50000
N_PAD = 50048
N_EDGES = 800000
EP = 802816
HID = 64
NUM_LAYERS = 10
NUM_GRAPHS = 16

_EB = 2048
_NB = 3128

_mesh_cache = []


def _mesh():
    if not _mesh_cache:
        _mesh_cache.append(plsc.VectorSubcoreMesh(
            core_axis_name="c", subcore_axis_name="s"))
    return _mesh_cache[0]


def _ln(x, g, b, eps=1e-5):
    m = jnp.mean(x, axis=-1, keepdims=True)
    v = jnp.mean((x - m) ** 2, axis=-1, keepdims=True)
    return (x - m) / jnp.sqrt(v + eps) * g + b


def _silu(x):
    return x * jax.nn.sigmoid(x)


def _softplus(x):
    return jnp.maximum(x, 0.0) + jnp.log1p(jnp.exp(-jnp.abs(x)))


def _dot(a, b):
    return jnp.dot(a, b, preferred_element_type=jnp.float32)


_OPS_W = EP // 128 // 32
_OPS_T = EP // 128 // 16
_TROWS = N_PAD // 16


def _sc_gather(table, idx2d):
    @functools.partial(
        pl.kernel, mesh=_mesh(),
        compiler_params=pltpu.CompilerParams(use_tc_tiling_on_sc=False),
        out_type=jax.ShapeDtypeStruct((EP, 64), jnp.float32),
        scratch_types=[
            pltpu.VMEM((_OPS_W, 128), jnp.int32),
            pltpu.VMEM((512, 64), jnp.float32),
            pltpu.VMEM((512, 64), jnp.float32),
            pltpu.SemaphoreType.DMA,
            pltpu.SemaphoreType.DMA,
            pltpu.SemaphoreType.DMA,
            pltpu.SemaphoreType.DMA,
        ],
    )
    def k(table_hbm, idx_hbm, out_hbm, idx_v, st0, st1, g0, g1, w0, w1):
        cid = lax.axis_index("c")
        sid = lax.axis_index("s")
        wid = sid * 2 + cid
        pltpu.sync_copy(idx_hbm.at[wid], idx_v)
        ebase = wid * _OPS_W * 128

        def fire4(g, stage, sem):
            for b in range(4):
                pltpu.async_copy(table_hbm.at[idx_v.at[g * 4 + b]],
                                 stage.at[pl.ds(b * 128, 128), :], sem)

        def wait4(stage, sem):
            pltpu.make_async_copy(table_hbm.at[idx_v.at[0]], stage, sem).wait()

        def wr(g, stage, sem):
            pltpu.async_copy(stage,
                             out_hbm.at[pl.ds(ebase + g * 512, 512), :], sem)

        def wrwait(stage, sem):
            pltpu.make_async_copy(
                stage, out_hbm.at[pl.ds(ebase, 512), :], sem).wait()

        fire4(0, st0, g0)

        def body(i, carry):
            fire4(2 * i + 1, st1, g1)
            wait4(st0, g0)
            wr(2 * i, st0, w0)
            wrwait(st0, w0)
            fire4(2 * i + 2, st0, g0)
            wait4(st1, g1)
            wr(2 * i + 1, st1, w1)
            wrwait(st1, w1)
            return carry

        lax.fori_loop(0, 24, body, 0)
        wait4(st0, g0)
        pltpu.sync_copy(st0, out_hbm.at[pl.ds(ebase + 48 * 512, 512), :])

    return k(table, idx2d)


def _sc_scatter_add(msg01, dst3s, zeros32):
    @functools.partial(
        pl.kernel, mesh=_mesh(),
        compiler_params=pltpu.CompilerParams(use_tc_tiling_on_sc=False),
        out_type=jax.ShapeDtypeStruct((2, N_PAD, 32), jnp.float32),
        scratch_types=[
            pltpu.VMEM((56, 128), jnp.int32),
            pltpu.VMEM((128, 32), jnp.float32),
            pltpu.VMEM((128, 32), jnp.float32),
            pltpu.VMEM((128, 32), jnp.float32),
            pltpu.VMEM((128, 32), jnp.float32),
            pltpu.VMEM_SHARED((N_PAD, 32), jnp.float32),
            pltpu.SemaphoreType.DMA,
            pltpu.SemaphoreType.DMA,
            pltpu.SemaphoreType.DMA,
            pltpu.SemaphoreType.DMA,
            pltpu.SemaphoreType.DMA,
            pltpu.SemaphoreType.DMA,
            pltpu.SemaphoreType.DMA,
            pltpu.SemaphoreType.DMA,
        ],
    )
    def k(msg_hbm, idx_hbm, z_hbm, out_hbm, idx_v, b0, b1, b2, b3, acc,
          l0, l1, l2, l3, s0, s1, s2, s3):
        cid = lax.axis_index("c")
        sid = lax.axis_index("s")
        bufs = (b0, b1, b2, b3)
        lsems = (l0, l1, l2, l3)
        ssems = (s0, s1, s2, s3)
        tbase = sid * _OPS_T * 128

        @pl.when(sid == 0)
        def _():
            pltpu.sync_copy(z_hbm, acc)

        plsc.subcore_barrier()

        def load(op, b, sem):
            pltpu.async_copy(
                msg_hbm.at[cid, pl.ds(tbase + op * 128, 128), :], bufs[b],
                sem)

        def loadwait(b, sem):
            pltpu.make_async_copy(
                msg_hbm.at[cid, pl.ds(tbase, 128), :], bufs[b], sem).wait()

        for b in range(4):
            load(b, b, lsems[b])

        def outer(m, carry):
            pltpu.sync_copy(idx_hbm.at[sid, pl.ds(m * 56, 56), :], idx_v)

            def inner(g, carry2):
                for b in range(4):
                    opl = g * 4 + b
                    op = m * 56 + opl
                    loadwait(b, lsems[b])
                    pltpu.async_copy(bufs[b], acc.at[idx_v.at[opl]],
                                     ssems[b], add=True)
                for b in range(4):
                    opn = m * 56 + g * 4 + b + 4
                    pltpu.make_async_copy(
                        bufs[b], acc.at[idx_v.at[0]], ssems[b]).wait()

                    @pl.when(opn < _OPS_T)
                    def _():
                        load(opn, b, lsems[b])

                return carry2

            return lax.fori_loop(0, 14, inner, carry)

        lax.fori_loop(0, 7, outer, 0)
        plsc.subcore_barrier()
        pltpu.sync_copy(acc.at[pl.ds(sid * _TROWS, _TROWS), :],
                        out_hbm.at[cid, pl.ds(sid * _TROWS, _TROWS), :])

    return k(msg01, dst3s, zeros32)


def _sc_count(ones128, dst3s, zeros32):
    @functools.partial(
        pl.kernel, mesh=_mesh(),
        compiler_params=pltpu.CompilerParams(use_tc_tiling_on_sc=False),
        out_type=jax.ShapeDtypeStruct((2, N_PAD, 32), jnp.float32),
        scratch_types=[
            pltpu.VMEM((56, 128), jnp.int32),
            pltpu.VMEM((128, 32), jnp.float32),
            pltpu.VMEM_SHARED((N_PAD, 32), jnp.float32),
            pltpu.SemaphoreType.DMA,
        ],
    )
    def k(ones_hbm, idx_hbm, z_hbm, out_hbm, idx_v, buf, acc, s0):
        cid = lax.axis_index("c")
        sid = lax.axis_index("s")

        @pl.when(sid == 0)
        def _():
            pltpu.sync_copy(z_hbm, acc)

        pltpu.sync_copy(ones_hbm, buf)
        plsc.subcore_barrier()

        def outer(m, carry):
            pltpu.sync_copy(idx_hbm.at[sid, pl.ds(m * 56, 56), :], idx_v)

            def inner(j, carry2):
                pltpu.async_copy(buf, acc.at[idx_v.at[j]], s0, add=True)
                pltpu.make_async_copy(buf, acc.at[idx_v.at[0]], s0).wait()
                return carry2

            return lax.fori_loop(0, 56, inner, carry)

        lax.fori_loop(0, 7, outer, 0)

        plsc.subcore_barrier()
        pltpu.sync_copy(acc.at[pl.ds(sid * _TROWS, _TROWS), :],
                        out_hbm.at[cid, pl.ds(sid * _TROWS, _TROWS), :])

    return k(ones128, dst3s, zeros32)


def _eblk(i):
    return (i, 0)


def _edge_specs(n):
    return [pl.BlockSpec((_EB, 64), _eblk) for _ in range(n)]


def _w_spec(shape):
    return pl.BlockSpec(shape, lambda i: tuple(0 for _ in shape))


def _split_spec():
    return pl.BlockSpec((2, _EB, 32), lambda i: (0, i, 0))


def _nsplit_spec():
    return pl.BlockSpec((2, _NB, 32), lambda i: (0, i, 0))


def _enc_body(x_ref, w, b, g, bb, out_ref):
    z = _ln(_dot(x_ref[...], w[...]) + b[...], g[...], bb[...])
    out_ref[...] = _silu(z)


def _encode_nodes(x_pad, p):
    return pl.pallas_call(
        _enc_body,
        grid=(N_PAD // _NB,),
        in_specs=[pl.BlockSpec((_NB, 4), _eblk), _w_spec((4, HID)),
                  _w_spec((1, HID)), _w_spec((1, HID)), _w_spec((1, HID))],
        out_specs=pl.BlockSpec((_NB, 64), _eblk),
        out_shape=jax.ShapeDtypeStruct((N_PAD, 64), jnp.float32),
    )(x_pad, p['lin']['W'], p['lin']['b'].reshape(1, -1),
      p['ln']['g'].reshape(1, -1), p['ln']['b'].reshape(1, -1))


def _encode_edges(ea_pad, p):
    return pl.pallas_call(
        _enc_body,
        grid=(EP // _EB,),
        in_specs=[pl.BlockSpec((_EB, 3), _eblk), _w_spec((3, HID)),
                  _w_spec((1, HID)), _w_spec((1, HID)), _w_spec((1, HID))],
        out_specs=pl.BlockSpec((_EB, 64), _eblk),
        out_shape=jax.ShapeDtypeStruct((EP, 64), jnp.float32),
    )(ea_pad, p['lin']['W'], p['lin']['b'].reshape(1, -1),
      p['ln']['g'].reshape(1, -1), p['ln']['b'].reshape(1, -1))


def _gine_edge_body(e_ref, hs_ref, w1, b1, g1, bb1, w2, b2,
                    eo_ref, msg_ref):
    ep = _silu(_ln(_dot(e_ref[...], w1[...]) + b1[...], g1[...], bb1[...]))
    eo_ref[...] = ep
    ee = _dot(ep, w2[...]) + b2[...]
    m = jnp.maximum(hs_ref[...] + ee, 0.0)
    msg_ref[0] = m[:, :32]
    msg_ref[1] = m[:, 32:]


def _gine_edge(e, hs, lp):
    gp = lp['gine']
    return pl.pallas_call(
        _gine_edge_body,
        grid=(EP // _EB,),
        in_specs=_edge_specs(2) + [
            _w_spec((64, 64)), _w_spec((1, 64)), _w_spec((1, 64)),
            _w_spec((1, 64)), _w_spec((64, 64)), _w_spec((1, 64))],
        out_specs=[pl.BlockSpec((_EB, 64), _eblk), _split_spec()],
        out_shape=[jax.ShapeDtypeStruct((EP, 64), jnp.float32),
                   jax.ShapeDtypeStruct((2, EP, 32), jnp.float32)],
    )(e, hs,
      lp['ec_lin']['W'], lp['ec_lin']['b'].reshape(1, -1),
      lp['ec_ln']['g'].reshape(1, -1), lp['ec_ln']['b'].reshape(1, -1),
      gp['lin_edge']['W'], gp['lin_edge']['b'].reshape(1, -1))


def _gine_node_body(h_ref, agg_ref, m1w, m1b, mlg, mlb, m2w, m2b,
                    ng, nb, pw, pb, pg, pbb, out_ref):
    h = h_ref[...]
    agg = jnp.concatenate([agg_ref[0], agg_ref[1]], axis=1)
    z = h + agg
    z = _silu(_ln(_dot(z, m1w[...]) + m1b[...], mlg[...], mlb[...]))
    z = _dot(z, m2w[...]) + m2b[...]
    hn = _ln(z, ng[...], nb[...])
    h2 = h + hn
    h2 = _silu(_ln(_dot(h2, pw[...]) + pb[...], pg[...], pbb[...]))
    out_ref[...] = h2


def _gine_node(h, agg01, lp, pp):
    gp = lp['gine']
    r2 = lambda v: v.reshape(1, -1)
    return pl.pallas_call(
        _gine_node_body,
        grid=(N_PAD // _NB,),
        in_specs=[pl.BlockSpec((_NB, 64), _eblk), _nsplit_spec(),
                  _w_spec((64, 64)), _w_spec((1, 64)), _w_spec((1, 64)),
                  _w_spec((1, 64)), _w_spec((64, 64)), _w_spec((1, 64)),
                  _w_spec((1, 64)), _w_spec((1, 64)),
                  _w_spec((64, 64)), _w_spec((1, 64)), _w_spec((1, 64)),
                  _w_spec((1, 64))],
        out_specs=pl.BlockSpec((_NB, 64), _eblk),
        out_shape=jax.ShapeDtypeStruct((N_PAD, 64), jnp.float32),
    )(h, agg01,
      gp['mlp1']['W'], r2(gp['mlp1']['b']), r2(gp['mlp_ln']['g']),
      r2(gp['mlp_ln']['b']), gp['mlp2']['W'], r2(gp['mlp2']['b']),
      r2(lp['norm']['g']), r2(lp['norm']['b']),
      pp['lin']['W'], r2(pp['lin']['b']), r2(pp['ln']['g']),
      r2(pp['ln']['b']))


def _qkv_body(h_ref, qw, qb, kw, kb, vw, vb, sw, sb,
              q_ref, k_ref, v_ref, s_ref):
    h = h_ref[...]
    q_ref[...] = _dot(h, qw[...]) + qb[...]
    k_ref[...] = _dot(h, kw[...]) + kb[...]
    v_ref[...] = _dot(h, vw[...]) + vb[...]
    s_ref[...] = _dot(h, sw[...]) + sb[...]


def _qkv(h, tp):
    r2 = lambda v: v.reshape(1, -1)
    outs = pl.pallas_call(
        _qkv_body,
        grid=(N_PAD // _NB,),
        in_specs=[pl.BlockSpec((_NB, 64), _eblk)] + [
            _w_spec((64, 64)) if i % 2 == 0 else _w_spec((1, 64))
            for i in range(8)],
        out_specs=[pl.BlockSpec((_NB, 64), _eblk) for _ in range(4)],
        out_shape=[jax.ShapeDtypeStruct((N_PAD, 64), jnp.float32)] * 4,
    )(h, tp['q']['W'], r2(tp['q']['b']), tp['k']['W'], r2(tp['k']['b']),
      tp['v']['W'], r2(tp['v']['b']), tp['skip']['W'], r2(tp['skip']['b']))
    return outs


def _tc_edge_body(e_ref, qd_ref, kk_ref, vv_ref, w1, b1, g1, bb1, w2, b2,
                  eo_ref, a_ref, mb_ref):
    ep = _silu(_ln(_dot(e_ref[...], w1[...]) + b1[...], g1[...], bb1[...]))
    eo_ref[...] = ep
    ee = _dot(ep, w2[...]) + b2[...]
    kj = kk_ref[...] + ee
    t = qd_ref[...] * kj
    lanes = lax.broadcasted_iota(jnp.int32, (64, 8), 0)
    heads = lax.broadcasted_iota(jnp.int32, (64, 8), 1)
    sel = (lanes // 8 == heads).astype(jnp.float32)
    a8 = _dot(t, sel) * (1.0 / np.sqrt(8.0))
    a64 = _dot(a8, sel.T)
    a_ref[0] = a64[:, :32]
    a_ref[1] = a64[:, 32:]
    mb = vv_ref[...] + ee
    mb_ref[0] = mb[:, :32]
    mb_ref[1] = mb[:, 32:]


def _tc_edge(e, qd, kk, vv, lp):
    tp = lp['tc']
    r2 = lambda v: v.reshape(1, -1)
    return pl.pallas_call(
        _tc_edge_body,
        grid=(EP // _EB,),
        in_specs=_edge_specs(4) + [
            _w_spec((64, 64)), _w_spec((1, 64)), _w_spec((1, 64)),
            _w_spec((1, 64)), _w_spec((64, 64)), _w_spec((1, 64))],
        out_specs=[pl.BlockSpec((_EB, 64), _eblk), _split_spec(),
                   _split_spec()],
        out_shape=[jax.ShapeDtypeStruct((EP, 64), jnp.float32),
                   jax.ShapeDtypeStruct((2, EP, 32), jnp.float32),
                   jax.ShapeDtypeStruct((2, EP, 32), jnp.float32)],
    )(e, qd, kk, vv,
      lp['ec_lin']['W'], r2(lp['ec_lin']['b']),
      r2(lp['ec_ln']['g']), r2(lp['ec_ln']['b']),
      tp['e']['W'], r2(tp['e']['b']))


def _cmean_body(cs_ref, cnt_ref, out_ref):
    cs = jnp.concatenate([cs_ref[0], cs_ref[1]], axis=1)
    cnt = cnt_ref[0][:, :1]
    out_ref[...] = jnp.where(cnt > 0.0, cs / jnp.maximum(cnt, 1.0), 0.0)


def _cmean(csum01, cnt01):
    return pl.pallas_call(
        _cmean_body,
        grid=(N_PAD // _NB,),
        in_specs=[_nsplit_spec(), _nsplit_spec()],
        out_specs=pl.BlockSpec((_NB, 64), _eblk),
        out_shape=jax.ShapeDtypeStruct((N_PAD, 64), jnp.float32),
    )(csum01, cnt01)


def _ex_body(a_ref, cd_ref, ex_ref):
    a = jnp.concatenate([a_ref[0], a_ref[1]], axis=1)
    ex = jnp.exp(a - cd_ref[...])
    ex_ref[0] = ex[:, :32]
    ex_ref[1] = ex[:, 32:]


def _ex(a01, cd):
    return pl.pallas_call(
        _ex_body,
        grid=(EP // _EB,),
        in_specs=[_split_spec(), pl.BlockSpec((_EB, 64), _eblk)],
        out_specs=_split_spec(),
        out_shape=jax.ShapeDtypeStruct((2, EP, 32), jnp.float32),
    )(a01, cd)


def _msg_body(ex_ref, mb_ref, dd_ref, msg_ref):
    ex = jnp.concatenate([ex_ref[0], ex_ref[1]], axis=1)
    mb = jnp.concatenate([mb_ref[0], mb_ref[1]], axis=1)
    a = ex / (dd_ref[...] + 1e-16)
    m = mb * a
    msg_ref[0] = m[:, :32]
    msg_ref[1] = m[:, 32:]


def _msg(ex01, mb01, dd):
    return pl.pallas_call(
        _msg_body,
        grid=(EP // _EB,),
        in_specs=[_split_spec(), _split_spec(),
                  pl.BlockSpec((_EB, 64), _eblk)],
        out_specs=_split_spec(),
        out_shape=jax.ShapeDtypeStruct((2, EP, 32), jnp.float32),
    )(ex01, mb01, dd)


def _gate_body(h_ref, out01_ref, xr_ref, wb, ng, nb, o_ref):
    h = h_ref[...]
    out = jnp.concatenate([out01_ref[0], out01_ref[1]], axis=1)
    xr = xr_ref[...]
    cat = jnp.concatenate([out, xr, out - xr], axis=1)
    beta = jax.nn.sigmoid(_dot(cat, wb[...]))
    hn = beta * xr + (1.0 - beta) * out
    hn = _ln(hn, ng[...], nb[...])
    o_ref[...] = h + hn


def _gate(h, out01, xr, lp):
    r2 = lambda v: v.reshape(1, -1)
    return pl.pallas_call(
        _gate_body,
        grid=(N_PAD // _NB,),
        in_specs=[pl.BlockSpec((_NB, 64), _eblk), _nsplit_spec(),
                  pl.BlockSpec((_NB, 64), _eblk),
                  _w_spec((192, 1)), _w_spec((1, 64)), _w_spec((1, 64))],
        out_specs=pl.BlockSpec((_NB, 64), _eblk),
        out_shape=jax.ShapeDtypeStruct((N_PAD, 64), jnp.float32),
    )(h, out01, xr, lp['tc']['Wbeta'],
      r2(lp['norm']['g']), r2(lp['norm']['b']))


_S2S_CHUNK = 2944
_S2S_NCHUNK = N_PAD // _S2S_CHUNK


def _s2s_head_body(*refs):
    (h_ref, oh_ref, gf_ref,
     wih0, whh0, bih0, bhh0,
     wih1, whh1, bih1, bhh1,
     rp_w, rp_b, rp_g, rp_b2,
     gm_w1, gm_b1, gm_g, gm_bln, gm_w2, gm_b2,
     fm_w1, fm_b1, fm_g1, fm_bln1,
     fm_w2, fm_b2, fm_g2, fm_bln2,
     fm_w3, fm_b3,
     out_ref) = refs

    B = NUM_GRAPHS
    H = HID
    CH = _S2S_CHUNK

    def one_s2s(wih, whh, bih, bhh):
        hst = jnp.zeros((B, H), jnp.float32)
        cst = jnp.zeros((B, H), jnp.float32)
        qs = jnp.zeros((B, 2 * H), jnp.float32)
        for _ in range(4):
            z = (_dot(qs, wih[...]) + bih[...]
                 + _dot(hst, whh[...]) + bhh[...])
            i_g, f_g, g_g, o_g = jnp.split(z, 4, axis=-1)
            cst = jax.nn.sigmoid(f_g) * cst + jax.nn.sigmoid(i_g) * jnp.tanh(g_g)
            hst = jax.nn.sigmoid(o_g) * jnp.tanh(cst)

            def max_step(c, emax):
                xc = h_ref[pl.ds(c * CH, CH), :]
                mc = oh_ref[:, pl.ds(c * CH, CH)]
                e0 = lax.dot_general(hst, xc, (((1,), (1,)), ((), ())),
                                     preferred_element_type=jnp.float32)
                emsk = jnp.where(mc > 0.0, e0, -jnp.inf)
                return jnp.maximum(emax, jnp.max(emsk, axis=1, keepdims=True))

            emax = lax.fori_loop(0, _S2S_NCHUNK, max_step,
                                 jnp.full((B, 1), -jnp.inf, jnp.float32))
            emax = jnp.where(jnp.isfinite(emax), emax, 0.0)

            def acc_step(c, carry):
                den, num = carry
                xc = h_ref[pl.ds(c * CH, CH), :]
                mc = oh_ref[:, pl.ds(c * CH, CH)]
                e0 = lax.dot_general(hst, xc, (((1,), (1,)), ((), ())),
                                     preferred_element_type=jnp.float32)
                eng = jnp.sum(e0 * mc, axis=0, keepdims=True)
                esel = jnp.sum(mc * emax, axis=0, keepdims=True)
                ex = jnp.exp(eng - esel)
                w = mc * ex
                den = den + jnp.sum(w, axis=1, keepdims=True)
                num = num + _dot(w, xc)
                return den, num

            den, num = lax.fori_loop(
                0, _S2S_NCHUNK, acc_step,
                (jnp.zeros((B, 1), jnp.float32),
                 jnp.zeros((B, H), jnp.float32)))
            r = num / (den + 1e-16)
            qs = jnp.concatenate([hst, r], axis=1)
        return qs

    qs0 = one_s2s(wih0, whh0, bih0, bhh0)
    qs1 = one_s2s(wih1, whh1, bih1, bhh1)
    hr = jnp.concatenate([qs0, qs1], axis=1)
    hr = _silu(_ln(_dot(hr, rp_w[...]) + rp_b[...], rp_g[...], rp_b2[...]))
    g = _silu(_ln(_dot(gf_ref[...], gm_w1[...]) + gm_b1[...],
                  gm_g[...], gm_bln[...]))
    g = _dot(g, gm_w2[...]) + gm_b2[...]
    comb = jnp.concatenate([hr, g], axis=1)
    f = _silu(_ln(_dot(comb, fm_w1[...]) + fm_b1[...], fm_g1[...],
                  fm_bln1[...]))
    f = _silu(_ln(_dot(f, fm_w2[...]) + fm_b2[...], fm_g2[...],
                  fm_bln2[...]))
    f = _dot(f, fm_w3[...]) + fm_b3[...]
    out_ref[...] = _softplus(f).T


def _s2s_head(h_pad, oh, gf, params):
    s2s = params['s2s']
    rp = params['rp']
    gm = params['gm']
    fm = params['fm']
    r2 = lambda v: v.reshape(1, -1)
    args = [
        h_pad, oh, gf,
        s2s[0]['Wih'], s2s[0]['Whh'], r2(s2s[0]['bih']), r2(s2s[0]['bhh']),
        s2s[1]['Wih'], s2s[1]['Whh'], r2(s2s[1]['bih']), r2(s2s[1]['bhh']),
        rp['lin']['W'], r2(rp['lin']['b']), r2(rp['ln']['g']), r2(rp['ln']['b']),
        gm['lin1']['W'], r2(gm['lin1']['b']), r2(gm['ln']['g']), r2(gm['ln']['b']),
        gm['lin2']['W'], r2(gm['lin2']['b']),
        fm['lin1']['W'], r2(fm['lin1']['b']), r2(fm['ln1']['g']), r2(fm['ln1']['b']),
        fm['lin2']['W'], r2(fm['lin2']['b']), r2(fm['ln2']['g']), r2(fm['ln2']['b']),
        fm['lin3']['W'], r2(fm['lin3']['b']),
    ]
    out = pl.pallas_call(
        _s2s_head_body,
        out_shape=jax.ShapeDtypeStruct((1, NUM_GRAPHS), jnp.float32),
    )(*args)
    return out.reshape(NUM_GRAPHS)


def kernel(x, edge_attr, nA, nB, system_size, params, edge_index, batch):
    src = edge_index[0]
    dst = edge_index[1]
    epad = EP - N_EDGES
    src3g = jnp.pad(src, (0, epad)).reshape(32, _OPS_W, 128)
    dst_p = jnp.pad(dst, (0, epad), constant_values=N_NODES)
    dst3g = dst_p.reshape(32, _OPS_W, 128)
    dst3s = dst_p.reshape(16, _OPS_T, 128)

    x_pad = jnp.pad(x, ((0, N_PAD - N_NODES), (0, 0)))
    ea_pad = jnp.pad(edge_attr, ((0, epad), (0, 0)))
    zeros32 = jnp.zeros((N_PAD, 32), jnp.float32)
    ones128 = jnp.ones((128, 32), jnp.float32)

    h = _encode_nodes(x_pad, params['node_enc'])
    e = _encode_edges(ea_pad, params['edge_enc'])

    cnt01 = _sc_count(ones128, dst3s, zeros32)

    for i in range(NUM_LAYERS):
        lp = params['layers'][i]
        if i % 2 == 0:
            hs = _sc_gather(h, src3g)
            e, msg01 = _gine_edge(e, hs, lp)
            agg01 = _sc_scatter_add(msg01, dst3s, zeros32)
            h = _gine_node(h, agg01, lp, params['pool'][i // 2])
        else:
            q, kk, vv, xr = _qkv(h, lp['tc'])
            qd = _sc_gather(q, dst3g)
            ks = _sc_gather(kk, src3g)
            vs = _sc_gather(vv, src3g)
            e, a01, mb01 = _tc_edge(e, qd, ks, vs, lp)
            csum01 = _sc_scatter_add(a01, dst3s, zeros32)
            c = _cmean(csum01, cnt01)
            cd = _sc_gather(c, dst3g)
            ex01 = _ex(a01, cd)
            den01 = _sc_scatter_add(ex01, dst3s, zeros32)
            den = jnp.concatenate([den01[0], den01[1]], axis=1)
            dd = _sc_gather(den, dst3g)
            msg01 = _msg(ex01, mb01, dd)
            out01 = _sc_scatter_add(msg01, dst3s, zeros32)
            h = _gate(h, out01, xr, lp)

    gids = jnp.arange(NUM_GRAPHS, dtype=jnp.int32)
    batch_pad = jnp.pad(batch, (0, N_PAD - N_NODES), constant_values=-1)
    oh = (batch_pad[None, :] == gids[:, None]).astype(jnp.float32)
    nAn = nA[:, 0] / (system_size[:, 0] + 1e-10)
    nBn = nB[:, 0] / (system_size[:, 0] + 1e-10)
    gf = jnp.stack([nAn, nBn], axis=1)
    return _s2s_head(h, oh, gf, params)


# speedup vs baseline: 17.2128x; 1.0769x over previous
import functools

import jax
import jax.numpy as jnp
import numpy as np
from jax import lax
from jax.experimental import pallas as pl
from jax.experimental.pallas import tpu as pltpu
from jax.experimental.pallas import tpu_sc as plsc

N_NODES = 50000
N_PAD = 50048
N_EDGES = 800000
EP = 802816
HID = 64
NUM_LAYERS = 10
NUM_GRAPHS = 16

_EB = 4096
_NB = 3128

_mesh_cache = []


def _mesh():
    if not _mesh_cache:
        _mesh_cache.append(plsc.VectorSubcoreMesh(
            core_axis_name="c", subcore_axis_name="s"))
    return _mesh_cache[0]


def _ln(x, g, b, eps=1e-5):
    m = jnp.mean(x, axis=-1, keepdims=True)
    v = jnp.mean((x - m) ** 2, axis=-1, keepdims=True)
    return (x - m) / jnp.sqrt(v + eps) * g + b


def _silu(x):
    return x * jax.nn.sigmoid(x)


def _softplus(x):
    return jnp.maximum(x, 0.0) + jnp.log1p(jnp.exp(-jnp.abs(x)))


def _dot(a, b):
    return jnp.dot(a, b, preferred_element_type=jnp.float32)


_OPS_W = EP // 128 // 32
_OPS_T = EP // 128 // 16
_TROWS = N_PAD // 16


def _sc_gather(table, idx2d):
    @functools.partial(
        pl.kernel, mesh=_mesh(),
        compiler_params=pltpu.CompilerParams(use_tc_tiling_on_sc=False),
        out_type=jax.ShapeDtypeStruct((EP, 64), jnp.float32),
        scratch_types=[
            pltpu.VMEM((_OPS_W, 128), jnp.int32),
            pltpu.VMEM((512, 64), jnp.float32),
            pltpu.VMEM((512, 64), jnp.float32),
            pltpu.SemaphoreType.DMA,
            pltpu.SemaphoreType.DMA,
            pltpu.SemaphoreType.DMA,
            pltpu.SemaphoreType.DMA,
        ],
    )
    def k(table_hbm, idx_hbm, out_hbm, idx_v, st0, st1, g0, g1, w0, w1):
        cid = lax.axis_index("c")
        sid = lax.axis_index("s")
        wid = sid * 2 + cid
        pltpu.sync_copy(idx_hbm.at[wid], idx_v)
        ebase = wid * _OPS_W * 128

        def fire4(g, stage, sem):
            for b in range(4):
                pltpu.async_copy(table_hbm.at[idx_v.at[g * 4 + b]],
                                 stage.at[pl.ds(b * 128, 128), :], sem)

        def wait4(stage, sem):
            pltpu.make_async_copy(table_hbm.at[idx_v.at[0]], stage, sem).wait()

        def wr(g, stage, sem):
            pltpu.async_copy(stage,
                             out_hbm.at[pl.ds(ebase + g * 512, 512), :], sem)

        def wrwait(stage, sem):
            pltpu.make_async_copy(
                stage, out_hbm.at[pl.ds(ebase, 512), :], sem).wait()

        fire4(0, st0, g0)

        def body(i, carry):
            fire4(2 * i + 1, st1, g1)
            wait4(st0, g0)
            wr(2 * i, st0, w0)
            wrwait(st0, w0)
            fire4(2 * i + 2, st0, g0)
            wait4(st1, g1)
            wr(2 * i + 1, st1, w1)
            wrwait(st1, w1)
            return carry

        lax.fori_loop(0, 24, body, 0)
        wait4(st0, g0)
        pltpu.sync_copy(st0, out_hbm.at[pl.ds(ebase + 48 * 512, 512), :])

    return k(table, idx2d)


def _sc_scatter_add(msg01, dst3s, zeros32):
    @functools.partial(
        pl.kernel, mesh=_mesh(),
        compiler_params=pltpu.CompilerParams(use_tc_tiling_on_sc=False),
        out_type=jax.ShapeDtypeStruct((2, N_PAD, 32), jnp.float32),
        scratch_types=[
            pltpu.VMEM((56, 128), jnp.int32),
            pltpu.VMEM((128, 32), jnp.float32),
            pltpu.VMEM((128, 32), jnp.float32),
            pltpu.VMEM((128, 32), jnp.float32),
            pltpu.VMEM((128, 32), jnp.float32),
            pltpu.VMEM_SHARED((N_PAD, 32), jnp.float32),
            pltpu.SemaphoreType.DMA,
            pltpu.SemaphoreType.DMA,
            pltpu.SemaphoreType.DMA,
            pltpu.SemaphoreType.DMA,
            pltpu.SemaphoreType.DMA,
            pltpu.SemaphoreType.DMA,
            pltpu.SemaphoreType.DMA,
            pltpu.SemaphoreType.DMA,
        ],
    )
    def k(msg_hbm, idx_hbm, z_hbm, out_hbm, idx_v, b0, b1, b2, b3, acc,
          l0, l1, l2, l3, s0, s1, s2, s3):
        cid = lax.axis_index("c")
        sid = lax.axis_index("s")
        bufs = (b0, b1, b2, b3)
        lsems = (l0, l1, l2, l3)
        ssems = (s0, s1, s2, s3)
        tbase = sid * _OPS_T * 128

        @pl.when(sid == 0)
        def _():
            pltpu.sync_copy(z_hbm, acc)

        plsc.subcore_barrier()

        def load(op, b, sem):
            pltpu.async_copy(
                msg_hbm.at[cid, pl.ds(tbase + op * 128, 128), :], bufs[b],
                sem)

        def loadwait(b, sem):
            pltpu.make_async_copy(
                msg_hbm.at[cid, pl.ds(tbase, 128), :], bufs[b], sem).wait()

        for b in range(4):
            load(b, b, lsems[b])

        def outer(m, carry):
            pltpu.sync_copy(idx_hbm.at[sid, pl.ds(m * 56, 56), :], idx_v)

            def inner(g, carry2):
                for b in range(4):
                    opl = g * 4 + b
                    op = m * 56 + opl
                    loadwait(b, lsems[b])
                    pltpu.async_copy(bufs[b], acc.at[idx_v.at[opl]],
                                     ssems[b], add=True)
                for b in range(4):
                    opn = m * 56 + g * 4 + b + 4
                    pltpu.make_async_copy(
                        bufs[b], acc.at[idx_v.at[0]], ssems[b]).wait()

                    @pl.when(opn < _OPS_T)
                    def _():
                        load(opn, b, lsems[b])

                return carry2

            return lax.fori_loop(0, 14, inner, carry)

        lax.fori_loop(0, 7, outer, 0)
        plsc.subcore_barrier()
        pltpu.sync_copy(acc.at[pl.ds(sid * _TROWS, _TROWS), :],
                        out_hbm.at[cid, pl.ds(sid * _TROWS, _TROWS), :])

    return k(msg01, dst3s, zeros32)


def _sc_count(ones128, dst3s, zeros32):
    @functools.partial(
        pl.kernel, mesh=_mesh(),
        compiler_params=pltpu.CompilerParams(use_tc_tiling_on_sc=False),
        out_type=jax.ShapeDtypeStruct((2, N_PAD, 32), jnp.float32),
        scratch_types=[
            pltpu.VMEM((56, 128), jnp.int32),
            pltpu.VMEM((128, 32), jnp.float32),
            pltpu.VMEM_SHARED((N_PAD, 32), jnp.float32),
            pltpu.SemaphoreType.DMA,
        ],
    )
    def k(ones_hbm, idx_hbm, z_hbm, out_hbm, idx_v, buf, acc, s0):
        cid = lax.axis_index("c")
        sid = lax.axis_index("s")

        @pl.when(sid == 0)
        def _():
            pltpu.sync_copy(z_hbm, acc)

        pltpu.sync_copy(ones_hbm, buf)
        plsc.subcore_barrier()

        def outer(m, carry):
            pltpu.sync_copy(idx_hbm.at[sid, pl.ds(m * 56, 56), :], idx_v)

            def inner(j, carry2):
                pltpu.async_copy(buf, acc.at[idx_v.at[j]], s0, add=True)
                pltpu.make_async_copy(buf, acc.at[idx_v.at[0]], s0).wait()
                return carry2

            return lax.fori_loop(0, 56, inner, carry)

        lax.fori_loop(0, 7, outer, 0)

        plsc.subcore_barrier()
        pltpu.sync_copy(acc.at[pl.ds(sid * _TROWS, _TROWS), :],
                        out_hbm.at[cid, pl.ds(sid * _TROWS, _TROWS), :])

    return k(ones128, dst3s, zeros32)


def _eblk(i):
    return (i, 0)


def _edge_specs(n):
    return [pl.BlockSpec((_EB, 64), _eblk) for _ in range(n)]


def _w_spec(shape):
    return pl.BlockSpec(shape, lambda i: tuple(0 for _ in shape))


def _split_spec():
    return pl.BlockSpec((2, _EB, 32), lambda i: (0, i, 0))


def _nsplit_spec():
    return pl.BlockSpec((2, _NB, 32), lambda i: (0, i, 0))


def _enc_body(xT_ref, w, b, g, bb, out_ref):
    z = lax.dot_general(xT_ref[...], w[...], (((0,), (0,)), ((), ())),
                        preferred_element_type=jnp.float32)
    z = _ln(z + b[...], g[...], bb[...])
    out_ref[...] = _silu(z)


_NBL = 2944


def _encode_nodes(xT_pad, p):
    return pl.pallas_call(
        _enc_body,
        grid=(N_PAD // _NBL,),
        in_specs=[pl.BlockSpec((4, _NBL), lambda i: (0, i)), _w_spec((4, HID)),
                  _w_spec((1, HID)), _w_spec((1, HID)), _w_spec((1, HID))],
        out_specs=pl.BlockSpec((_NBL, 64), _eblk),
        out_shape=jax.ShapeDtypeStruct((N_PAD, 64), jnp.float32),
    )(xT_pad, p['lin']['W'], p['lin']['b'].reshape(1, -1),
      p['ln']['g'].reshape(1, -1), p['ln']['b'].reshape(1, -1))


def _encode_edges(eaT_pad, p):
    return pl.pallas_call(
        _enc_body,
        grid=(EP // _EB,),
        in_specs=[pl.BlockSpec((3, _EB), lambda i: (0, i)), _w_spec((3, HID)),
                  _w_spec((1, HID)), _w_spec((1, HID)), _w_spec((1, HID))],
        out_specs=pl.BlockSpec((_EB, 64), _eblk),
        out_shape=jax.ShapeDtypeStruct((EP, 64), jnp.float32),
    )(eaT_pad, p['lin']['W'], p['lin']['b'].reshape(1, -1),
      p['ln']['g'].reshape(1, -1), p['ln']['b'].reshape(1, -1))


def _gine_edge_body(e_ref, hs_ref, w1, b1, g1, bb1, w2, b2,
                    eo_ref, msg_ref):
    ep = _silu(_ln(_dot(e_ref[...], w1[...]) + b1[...], g1[...], bb1[...]))
    eo_ref[...] = ep
    ee = _dot(ep, w2[...]) + b2[...]
    m = jnp.maximum(hs_ref[...] + ee, 0.0)
    msg_ref[0] = m[:, :32]
    msg_ref[1] = m[:, 32:]


def _gine_edge(e, hs, lp):
    gp = lp['gine']
    return pl.pallas_call(
        _gine_edge_body,
        grid=(EP // _EB,),
        in_specs=_edge_specs(2) + [
            _w_spec((64, 64)), _w_spec((1, 64)), _w_spec((1, 64)),
            _w_spec((1, 64)), _w_spec((64, 64)), _w_spec((1, 64))],
        out_specs=[pl.BlockSpec((_EB, 64), _eblk), _split_spec()],
        out_shape=[jax.ShapeDtypeStruct((EP, 64), jnp.float32),
                   jax.ShapeDtypeStruct((2, EP, 32), jnp.float32)],
    )(e, hs,
      lp['ec_lin']['W'], lp['ec_lin']['b'].reshape(1, -1),
      lp['ec_ln']['g'].reshape(1, -1), lp['ec_ln']['b'].reshape(1, -1),
      gp['lin_edge']['W'], gp['lin_edge']['b'].reshape(1, -1))


def _gine_node_body(h_ref, agg_ref, m1w, m1b, mlg, mlb, m2w, m2b,
                    ng, nb, pw, pb, pg, pbb, out_ref):
    h = h_ref[...]
    agg = jnp.concatenate([agg_ref[0], agg_ref[1]], axis=1)
    z = h + agg
    z = _silu(_ln(_dot(z, m1w[...]) + m1b[...], mlg[...], mlb[...]))
    z = _dot(z, m2w[...]) + m2b[...]
    hn = _ln(z, ng[...], nb[...])
    h2 = h + hn
    h2 = _silu(_ln(_dot(h2, pw[...]) + pb[...], pg[...], pbb[...]))
    out_ref[...] = h2


def _gine_node(h, agg01, lp, pp):
    gp = lp['gine']
    r2 = lambda v: v.reshape(1, -1)
    return pl.pallas_call(
        _gine_node_body,
        grid=(N_PAD // _NB,),
        in_specs=[pl.BlockSpec((_NB, 64), _eblk), _nsplit_spec(),
                  _w_spec((64, 64)), _w_spec((1, 64)), _w_spec((1, 64)),
                  _w_spec((1, 64)), _w_spec((64, 64)), _w_spec((1, 64)),
                  _w_spec((1, 64)), _w_spec((1, 64)),
                  _w_spec((64, 64)), _w_spec((1, 64)), _w_spec((1, 64)),
                  _w_spec((1, 64))],
        out_specs=pl.BlockSpec((_NB, 64), _eblk),
        out_shape=jax.ShapeDtypeStruct((N_PAD, 64), jnp.float32),
    )(h, agg01,
      gp['mlp1']['W'], r2(gp['mlp1']['b']), r2(gp['mlp_ln']['g']),
      r2(gp['mlp_ln']['b']), gp['mlp2']['W'], r2(gp['mlp2']['b']),
      r2(lp['norm']['g']), r2(lp['norm']['b']),
      pp['lin']['W'], r2(pp['lin']['b']), r2(pp['ln']['g']),
      r2(pp['ln']['b']))


def _qkv_body(h_ref, qw, qb, kw, kb, vw, vb, sw, sb,
              q_ref, k_ref, v_ref, s_ref):
    h = h_ref[...]
    q_ref[...] = _dot(h, qw[...]) + qb[...]
    k_ref[...] = _dot(h, kw[...]) + kb[...]
    v_ref[...] = _dot(h, vw[...]) + vb[...]
    s_ref[...] = _dot(h, sw[...]) + sb[...]


def _qkv(h, tp):
    r2 = lambda v: v.reshape(1, -1)
    outs = pl.pallas_call(
        _qkv_body,
        grid=(N_PAD // _NB,),
        in_specs=[pl.BlockSpec((_NB, 64), _eblk)] + [
            _w_spec((64, 64)) if i % 2 == 0 else _w_spec((1, 64))
            for i in range(8)],
        out_specs=[pl.BlockSpec((_NB, 64), _eblk) for _ in range(4)],
        out_shape=[jax.ShapeDtypeStruct((N_PAD, 64), jnp.float32)] * 4,
    )(h, tp['q']['W'], r2(tp['q']['b']), tp['k']['W'], r2(tp['k']['b']),
      tp['v']['W'], r2(tp['v']['b']), tp['skip']['W'], r2(tp['skip']['b']))
    return outs


def _tc_edge_body(e_ref, qd_ref, kk_ref, vv_ref, w1, b1, g1, bb1, w2, b2,
                  eo_ref, a_ref, mb_ref):
    ep = _silu(_ln(_dot(e_ref[...], w1[...]) + b1[...], g1[...], bb1[...]))
    eo_ref[...] = ep
    ee = _dot(ep, w2[...]) + b2[...]
    kj = kk_ref[...] + ee
    t = qd_ref[...] * kj
    lanes = lax.broadcasted_iota(jnp.int32, (64, 8), 0)
    heads = lax.broadcasted_iota(jnp.int32, (64, 8), 1)
    sel = (lanes // 8 == heads).astype(jnp.float32)
    a8 = _dot(t, sel) * (1.0 / np.sqrt(8.0))
    a64 = _dot(a8, sel.T)
    a_ref[0] = a64[:, :32]
    a_ref[1] = a64[:, 32:]
    mb = vv_ref[...] + ee
    mb_ref[0] = mb[:, :32]
    mb_ref[1] = mb[:, 32:]


def _tc_edge(e, qd, kk, vv, lp):
    tp = lp['tc']
    r2 = lambda v: v.reshape(1, -1)
    return pl.pallas_call(
        _tc_edge_body,
        grid=(EP // _EB,),
        in_specs=_edge_specs(4) + [
            _w_spec((64, 64)), _w_spec((1, 64)), _w_spec((1, 64)),
            _w_spec((1, 64)), _w_spec((64, 64)), _w_spec((1, 64))],
        out_specs=[pl.BlockSpec((_EB, 64), _eblk), _split_spec(),
                   _split_spec()],
        out_shape=[jax.ShapeDtypeStruct((EP, 64), jnp.float32),
                   jax.ShapeDtypeStruct((2, EP, 32), jnp.float32),
                   jax.ShapeDtypeStruct((2, EP, 32), jnp.float32)],
    )(e, qd, kk, vv,
      lp['ec_lin']['W'], r2(lp['ec_lin']['b']),
      r2(lp['ec_ln']['g']), r2(lp['ec_ln']['b']),
      tp['e']['W'], r2(tp['e']['b']))


def _cmean_body(cs_ref, cnt_ref, out_ref):
    cs = jnp.concatenate([cs_ref[0], cs_ref[1]], axis=1)
    cnt = cnt_ref[0][:, :1]
    out_ref[...] = jnp.where(cnt > 0.0, cs / jnp.maximum(cnt, 1.0), 0.0)


def _cmean(csum01, cnt01):
    return pl.pallas_call(
        _cmean_body,
        grid=(N_PAD // _NB,),
        in_specs=[_nsplit_spec(), _nsplit_spec()],
        out_specs=pl.BlockSpec((_NB, 64), _eblk),
        out_shape=jax.ShapeDtypeStruct((N_PAD, 64), jnp.float32),
    )(csum01, cnt01)


def _ex_body(a_ref, cd_ref, ex_ref):
    a = jnp.concatenate([a_ref[0], a_ref[1]], axis=1)
    ex = jnp.exp(a - cd_ref[...])
    ex_ref[0] = ex[:, :32]
    ex_ref[1] = ex[:, 32:]


def _ex(a01, cd):
    return pl.pallas_call(
        _ex_body,
        grid=(EP // _EB,),
        in_specs=[_split_spec(), pl.BlockSpec((_EB, 64), _eblk)],
        out_specs=_split_spec(),
        out_shape=jax.ShapeDtypeStruct((2, EP, 32), jnp.float32),
    )(a01, cd)


def _msg_body(ex_ref, mb_ref, dd_ref, msg_ref):
    ex = jnp.concatenate([ex_ref[0], ex_ref[1]], axis=1)
    mb = jnp.concatenate([mb_ref[0], mb_ref[1]], axis=1)
    a = ex / (dd_ref[...] + 1e-16)
    m = mb * a
    msg_ref[0] = m[:, :32]
    msg_ref[1] = m[:, 32:]


def _msg(ex01, mb01, dd):
    return pl.pallas_call(
        _msg_body,
        grid=(EP // _EB,),
        in_specs=[_split_spec(), _split_spec(),
                  pl.BlockSpec((_EB, 64), _eblk)],
        out_specs=_split_spec(),
        out_shape=jax.ShapeDtypeStruct((2, EP, 32), jnp.float32),
    )(ex01, mb01, dd)


def _gate_body(h_ref, out01_ref, xr_ref, wb, ng, nb, o_ref):
    h = h_ref[...]
    out = jnp.concatenate([out01_ref[0], out01_ref[1]], axis=1)
    xr = xr_ref[...]
    cat = jnp.concatenate([out, xr, out - xr], axis=1)
    beta = jax.nn.sigmoid(_dot(cat, wb[...]))
    hn = beta * xr + (1.0 - beta) * out
    hn = _ln(hn, ng[...], nb[...])
    o_ref[...] = h + hn


def _gate(h, out01, xr, lp):
    r2 = lambda v: v.reshape(1, -1)
    return pl.pallas_call(
        _gate_body,
        grid=(N_PAD // _NB,),
        in_specs=[pl.BlockSpec((_NB, 64), _eblk), _nsplit_spec(),
                  pl.BlockSpec((_NB, 64), _eblk),
                  _w_spec((192, 1)), _w_spec((1, 64)), _w_spec((1, 64))],
        out_specs=pl.BlockSpec((_NB, 64), _eblk),
        out_shape=jax.ShapeDtypeStruct((N_PAD, 64), jnp.float32),
    )(h, out01, xr, lp['tc']['Wbeta'],
      r2(lp['norm']['g']), r2(lp['norm']['b']))


_S2S_CHUNK = 2944
_S2S_NCHUNK = N_PAD // _S2S_CHUNK


def _s2s_head_body(*refs):
    (h_ref, oh_ref, gf_ref,
     wih0, whh0, bih0, bhh0,
     wih1, whh1, bih1, bhh1,
     rp_w, rp_b, rp_g, rp_b2,
     gm_w1, gm_b1, gm_g, gm_bln, gm_w2, gm_b2,
     fm_w1, fm_b1, fm_g1, fm_bln1,
     fm_w2, fm_b2, fm_g2, fm_bln2,
     fm_w3, fm_b3,
     out_ref) = refs

    B = NUM_GRAPHS
    H = HID
    CH = _S2S_CHUNK

    def one_s2s(wih, whh, bih, bhh):
        hst = jnp.zeros((B, H), jnp.float32)
        cst = jnp.zeros((B, H), jnp.float32)
        qs = jnp.zeros((B, 2 * H), jnp.float32)
        for _ in range(4):
            z = (_dot(qs, wih[...]) + bih[...]
                 + _dot(hst, whh[...]) + bhh[...])
            i_g, f_g, g_g, o_g = jnp.split(z, 4, axis=-1)
            cst = jax.nn.sigmoid(f_g) * cst + jax.nn.sigmoid(i_g) * jnp.tanh(g_g)
            hst = jax.nn.sigmoid(o_g) * jnp.tanh(cst)

            def max_step(c, emax):
                xc = h_ref[pl.ds(c * CH, CH), :]
                mc = oh_ref[:, pl.ds(c * CH, CH)]
                e0 = lax.dot_general(hst, xc, (((1,), (1,)), ((), ())),
                                     preferred_element_type=jnp.float32)
                emsk = jnp.where(mc > 0.0, e0, -jnp.inf)
                return jnp.maximum(emax, jnp.max(emsk, axis=1, keepdims=True))

            emax = lax.fori_loop(0, _S2S_NCHUNK, max_step,
                                 jnp.full((B, 1), -jnp.inf, jnp.float32))
            emax = jnp.where(jnp.isfinite(emax), emax, 0.0)

            def acc_step(c, carry):
                den, num = carry
                xc = h_ref[pl.ds(c * CH, CH), :]
                mc = oh_ref[:, pl.ds(c * CH, CH)]
                e0 = lax.dot_general(hst, xc, (((1,), (1,)), ((), ())),
                                     preferred_element_type=jnp.float32)
                eng = jnp.sum(e0 * mc, axis=0, keepdims=True)
                esel = jnp.sum(mc * emax, axis=0, keepdims=True)
                ex = jnp.exp(eng - esel)
                w = mc * ex
                den = den + jnp.sum(w, axis=1, keepdims=True)
                num = num + _dot(w, xc)
                return den, num

            den, num = lax.fori_loop(
                0, _S2S_NCHUNK, acc_step,
                (jnp.zeros((B, 1), jnp.float32),
                 jnp.zeros((B, H), jnp.float32)))
            r = num / (den + 1e-16)
            qs = jnp.concatenate([hst, r], axis=1)
        return qs

    qs0 = one_s2s(wih0, whh0, bih0, bhh0)
    qs1 = one_s2s(wih1, whh1, bih1, bhh1)
    hr = jnp.concatenate([qs0, qs1], axis=1)
    hr = _silu(_ln(_dot(hr, rp_w[...]) + rp_b[...], rp_g[...], rp_b2[...]))
    g = _silu(_ln(_dot(gf_ref[...], gm_w1[...]) + gm_b1[...],
                  gm_g[...], gm_bln[...]))
    g = _dot(g, gm_w2[...]) + gm_b2[...]
    comb = jnp.concatenate([hr, g], axis=1)
    f = _silu(_ln(_dot(comb, fm_w1[...]) + fm_b1[...], fm_g1[...],
                  fm_bln1[...]))
    f = _silu(_ln(_dot(f, fm_w2[...]) + fm_b2[...], fm_g2[...],
                  fm_bln2[...]))
    f = _dot(f, fm_w3[...]) + fm_b3[...]
    out_ref[...] = _softplus(f).T


def _s2s_head(h_pad, oh, gf, params):
    s2s = params['s2s']
    rp = params['rp']
    gm = params['gm']
    fm = params['fm']
    r2 = lambda v: v.reshape(1, -1)
    args = [
        h_pad, oh, gf,
        s2s[0]['Wih'], s2s[0]['Whh'], r2(s2s[0]['bih']), r2(s2s[0]['bhh']),
        s2s[1]['Wih'], s2s[1]['Whh'], r2(s2s[1]['bih']), r2(s2s[1]['bhh']),
        rp['lin']['W'], r2(rp['lin']['b']), r2(rp['ln']['g']), r2(rp['ln']['b']),
        gm['lin1']['W'], r2(gm['lin1']['b']), r2(gm['ln']['g']), r2(gm['ln']['b']),
        gm['lin2']['W'], r2(gm['lin2']['b']),
        fm['lin1']['W'], r2(fm['lin1']['b']), r2(fm['ln1']['g']), r2(fm['ln1']['b']),
        fm['lin2']['W'], r2(fm['lin2']['b']), r2(fm['ln2']['g']), r2(fm['ln2']['b']),
        fm['lin3']['W'], r2(fm['lin3']['b']),
    ]
    out = pl.pallas_call(
        _s2s_head_body,
        out_shape=jax.ShapeDtypeStruct((1, NUM_GRAPHS), jnp.float32),
    )(*args)
    return out.reshape(NUM_GRAPHS)


def kernel(x, edge_attr, nA, nB, system_size, params, edge_index, batch):
    src = edge_index[0]
    dst = edge_index[1]
    epad = EP - N_EDGES
    src3g = jnp.pad(src, (0, epad)).reshape(32, _OPS_W, 128)
    dst_p = jnp.pad(dst, (0, epad), constant_values=N_NODES)
    dst3g = dst_p.reshape(32, _OPS_W, 128)
    dst3s = dst_p.reshape(16, _OPS_T, 128)

    xT_pad = jnp.pad(x.T, ((0, 0), (0, N_PAD - N_NODES)))
    eaT_pad = jnp.pad(edge_attr.T, ((0, 0), (0, epad)))
    zeros32 = jnp.zeros((N_PAD, 32), jnp.float32)
    ones128 = jnp.ones((128, 32), jnp.float32)

    h = _encode_nodes(xT_pad, params['node_enc'])
    e = _encode_edges(eaT_pad, params['edge_enc'])

    cnt01 = _sc_count(ones128, dst3s, zeros32)

    for i in range(NUM_LAYERS):
        lp = params['layers'][i]
        if i % 2 == 0:
            hs = _sc_gather(h, src3g)
            e, msg01 = _gine_edge(e, hs, lp)
            agg01 = _sc_scatter_add(msg01, dst3s, zeros32)
            h = _gine_node(h, agg01, lp, params['pool'][i // 2])
        else:
            q, kk, vv, xr = _qkv(h, lp['tc'])
            qd = _sc_gather(q, dst3g)
            ks = _sc_gather(kk, src3g)
            vs = _sc_gather(vv, src3g)
            e, a01, mb01 = _tc_edge(e, qd, ks, vs, lp)
            csum01 = _sc_scatter_add(a01, dst3s, zeros32)
            c = _cmean(csum01, cnt01)
            cd = _sc_gather(c, dst3g)
            ex01 = _ex(a01, cd)
            den01 = _sc_scatter_add(ex01, dst3s, zeros32)
            den = jnp.concatenate([den01[0], den01[1]], axis=1)
            dd = _sc_gather(den, dst3g)
            msg01 = _msg(ex01, mb01, dd)
            out01 = _sc_scatter_add(msg01, dst3s, zeros32)
            h = _gate(h, out01, xr, lp)

    gids = jnp.arange(NUM_GRAPHS, dtype=jnp.int32)
    batch_pad = jnp.pad(batch, (0, N_PAD - N_NODES), constant_values=-1)
    oh = (batch_pad[None, :] == gids[:, None]).astype(jnp.float32)
    nAn = nA[:, 0] / (system_size[:, 0] + 1e-10)
    nBn = nB[:, 0] / (system_size[:, 0] + 1e-10)
    gf = jnp.stack([nAn, nBn], axis=1)
    return _s2s_head(h, oh, gf, params)
